# Initial kernel scaffold; baseline (speedup 1.0000x reference)
import functools

import jax
import jax.numpy as jnp
import numpy as np
from jax import lax
from jax.experimental import pallas as pl
from jax.experimental.pallas import tpu as pltpu
from jax.experimental.pallas import tpu_sc as plsc

N = 10000
E = 320000
D = 128
HID = 128
HEADS = 8
NG = 64

NC = 2
NS = 16
NW = NC * NS
E_PER_W = E // NW
CH = 80
NCH = E_PER_W // CH
ROWS_A = 632
ROWS_LAST = N - 15 * ROWS_A

BN_BLK = 1000
GRID_N = N // BN_BLK


def _dense_block(x, W, Bs, Bd, skip_W=None, skip_b=None):
  D_in = x.shape[1]
  with_skip = skip_W is not None

  def body(*refs):
    if with_skip:
      (x_ref, w_ref, bs_ref, bd_ref, sw_ref, sb_ref,
       h_ref, as_ref, ad_ref, ms_ref, md_ref, sk_ref) = refs
    else:
      (x_ref, w_ref, bs_ref, bd_ref,
       h_ref, as_ref, ad_ref, ms_ref, md_ref) = refs
    i = pl.program_id(0)
    xb = x_ref[...]
    hb = jnp.dot(xb, w_ref[...], preferred_element_type=jnp.float32)
    h_ref[...] = hb
    a_s = jnp.dot(hb, bs_ref[...], preferred_element_type=jnp.float32)
    a_d = jnp.dot(hb, bd_ref[...], preferred_element_type=jnp.float32)
    as_ref[...] = a_s
    ad_ref[...] = a_d

    @pl.when(i == 0)
    def _():
      ms_ref[...] = jnp.full((1, 16), -1e30, jnp.float32)
      md_ref[...] = jnp.full((1, 16), -1e30, jnp.float32)

    ms_ref[...] = jnp.maximum(ms_ref[...], jnp.max(a_s, axis=0, keepdims=True))
    md_ref[...] = jnp.maximum(md_ref[...], jnp.max(a_d, axis=0, keepdims=True))
    if with_skip:
      sk_ref[...] = (jnp.dot(xb, sw_ref[...], preferred_element_type=jnp.float32)
                     + sb_ref[...])

  out_shape = [
      jax.ShapeDtypeStruct((N, HEADS * HID), jnp.float32),
      jax.ShapeDtypeStruct((N, 16), jnp.float32),
      jax.ShapeDtypeStruct((N, 16), jnp.float32),
      jax.ShapeDtypeStruct((1, 16), jnp.float32),
      jax.ShapeDtypeStruct((1, 16), jnp.float32),
  ]
  out_specs = [
      pl.BlockSpec((BN_BLK, HEADS * HID), lambda i: (i, 0)),
      pl.BlockSpec((BN_BLK, 16), lambda i: (i, 0)),
      pl.BlockSpec((BN_BLK, 16), lambda i: (i, 0)),
      pl.BlockSpec((1, 16), lambda i: (0, 0)),
      pl.BlockSpec((1, 16), lambda i: (0, 0)),
  ]
  in_specs = [
      pl.BlockSpec((BN_BLK, D_in), lambda i: (i, 0)),
      pl.BlockSpec((D_in, HEADS * HID), lambda i: (0, 0)),
      pl.BlockSpec((HEADS * HID, 16), lambda i: (0, 0)),
      pl.BlockSpec((HEADS * HID, 16), lambda i: (0, 0)),
  ]
  args = [x, W, Bs, Bd]
  if with_skip:
    in_specs += [pl.BlockSpec((D_in, HID), lambda i: (0, 0)),
                 pl.BlockSpec((1, HID), lambda i: (0, 0))]
    args += [skip_W, skip_b.reshape(1, HID)]
    out_shape.append(jax.ShapeDtypeStruct((N, HID), jnp.float32))
    out_specs.append(pl.BlockSpec((BN_BLK, HID), lambda i: (i, 0)))

  return pl.pallas_call(
      body, grid=(GRID_N,), in_specs=in_specs, out_specs=out_specs,
      out_shape=out_shape)(*args)


def _erf(x):
  a1, a2, a3, a4, a5 = (0.254829592, -0.284496736, 1.421413741,
                        -1.453152027, 1.061405429)
  p = 0.3275911
  s = jnp.sign(x)
  ax = jnp.abs(x)
  t = 1.0 / (1.0 + p * ax)
  poly = ((((a5 * t + a4) * t + a3) * t + a2) * t + a1) * t
  y = 1.0 - poly * jnp.exp(-ax * ax)
  return s * y


def _gelu(x):
  return x * 0.5 * (1.0 + _erf(x * np.float32(1.0 / np.sqrt(2.0))))


def _post_bn_gelu(p0, p1, skip, bias, g, b):
  def body(p0_ref, p1_ref, sk_ref, bias_ref, g_ref, b_ref, out_ref,
           sums, sumsq):
    ph = pl.program_id(0)
    i = pl.program_id(1)
    gv = (p0_ref[...] + p1_ref[...]) * np.float32(1.0 / HEADS) + bias_ref[...]

    @pl.when(ph == 0)
    def _():
      @pl.when(i == 0)
      def _():
        sums[...] = jnp.zeros((1, HID), jnp.float32)
        sumsq[...] = jnp.zeros((1, HID), jnp.float32)
      sums[...] += jnp.sum(gv, axis=0, keepdims=True)
      sumsq[...] += jnp.sum(gv * gv, axis=0, keepdims=True)

    @pl.when(ph == 1)
    def _():
      mu = sums[...] * np.float32(1.0 / N)
      var = sumsq[...] * np.float32(1.0 / N) - mu * mu
      xn = (gv - mu) * lax.rsqrt(var + 1e-5) * g_ref[...] + b_ref[...]
      out_ref[...] = _gelu(xn + sk_ref[...])

  return pl.pallas_call(
      body, grid=(2, GRID_N),
      in_specs=[
          pl.BlockSpec((BN_BLK, HID), lambda p, i: (i, 0)),
          pl.BlockSpec((BN_BLK, HID), lambda p, i: (i, 0)),
          pl.BlockSpec((BN_BLK, HID), lambda p, i: (i, 0)),
          pl.BlockSpec((1, HID), lambda p, i: (0, 0)),
          pl.BlockSpec((1, HID), lambda p, i: (0, 0)),
          pl.BlockSpec((1, HID), lambda p, i: (0, 0)),
      ],
      out_specs=pl.BlockSpec((BN_BLK, HID), lambda p, i: (i, 0)),
      out_shape=jax.ShapeDtypeStruct((N, HID), jnp.float32),
      scratch_shapes=[pltpu.VMEM((1, HID), jnp.float32),
                      pltpu.VMEM((1, HID), jnp.float32)],
  )(p0, p1, skip, bias.reshape(1, HID), g.reshape(1, HID), b.reshape(1, HID))


def _post_bn_gelu_pool(p0, p1, batch_row, bias, g, b):
  def body(p0_ref, p1_ref, bi_ref, bias_ref, g_ref, b_ref, out_ref,
           sums, sumsq, pool, cnt):
    ph = pl.program_id(0)
    i = pl.program_id(1)
    gv = (p0_ref[...] + p1_ref[...]) * np.float32(1.0 / HEADS) + bias_ref[...]

    @pl.when(ph == 0)
    def _():
      @pl.when(i == 0)
      def _():
        sums[...] = jnp.zeros((1, HID), jnp.float32)
        sumsq[...] = jnp.zeros((1, HID), jnp.float32)
      sums[...] += jnp.sum(gv, axis=0, keepdims=True)
      sumsq[...] += jnp.sum(gv * gv, axis=0, keepdims=True)

    @pl.when(ph == 1)
    def _():
      mu = sums[...] * np.float32(1.0 / N)
      var = sumsq[...] * np.float32(1.0 / N) - mu * mu
      xn = (gv - mu) * lax.rsqrt(var + 1e-5) * g_ref[...] + b_ref[...] + gv
      o = _gelu(xn)
      ids = lax.broadcasted_iota(jnp.int32, (NG, BN_BLK), 0)
      oh = (ids == bi_ref[...]).astype(jnp.float32)

      @pl.when(i == 0)
      def _():
        pool[...] = jnp.zeros((NG, HID), jnp.float32)
        cnt[...] = jnp.zeros((NG, HID), jnp.float32)

      pool[...] += lax.dot_general(oh, o, (((1,), (0,)), ((), ())),
                                   preferred_element_type=jnp.float32)
      cnt[...] += jnp.broadcast_to(jnp.sum(oh, axis=1, keepdims=True),
                                   (NG, HID))

      @pl.when(i == GRID_N - 1)
      def _():
        out_ref[...] = pool[...] / jnp.maximum(cnt[...], 1.0)

  return pl.pallas_call(
      body, grid=(2, GRID_N),
      in_specs=[
          pl.BlockSpec((BN_BLK, HID), lambda p, i: (i, 0)),
          pl.BlockSpec((BN_BLK, HID), lambda p, i: (i, 0)),
          pl.BlockSpec((1, BN_BLK), lambda p, i: (0, i)),
          pl.BlockSpec((1, HID), lambda p, i: (0, 0)),
          pl.BlockSpec((1, HID), lambda p, i: (0, 0)),
          pl.BlockSpec((1, HID), lambda p, i: (0, 0)),
      ],
      out_specs=pl.BlockSpec((NG, HID), lambda p, i: (0, 0)),
      out_shape=jax.ShapeDtypeStruct((NG, HID), jnp.float32),
      scratch_shapes=[pltpu.VMEM((1, HID), jnp.float32),
                      pltpu.VMEM((1, HID), jnp.float32),
                      pltpu.VMEM((NG, HID), jnp.float32),
                      pltpu.VMEM((NG, HID), jnp.float32)],
  )(p0, p1, batch_row, bias.reshape(1, HID), g.reshape(1, HID),
    b.reshape(1, HID))


def _sc_mesh():
  return plsc.VectorSubcoreMesh(core_axis_name="c", subcore_axis_name="s")


def _sc_pass1(aas, aad, m_raw, src, dst):
  @functools.partial(
      pl.kernel,
      out_type=(jax.ShapeDtypeStruct((E, 16), jnp.float32),
                jax.ShapeDtypeStruct((N, 32), jnp.float32)),
      mesh=_sc_mesh(),
      scratch_types=[
          pltpu.VMEM((CH,), jnp.int32),
          pltpu.VMEM((CH,), jnp.int32),
          pltpu.VMEM((CH, 16), jnp.float32),
          pltpu.VMEM((CH, 16), jnp.float32),
          pltpu.VMEM((CH, 16), jnp.float32),
          pltpu.VMEM((16,), jnp.float32),
          pltpu.VMEM((ROWS_A, 16), jnp.float32),
          pltpu.VMEM_SHARED((N, 16), jnp.float32),
          pltpu.SemaphoreType.DMA,
      ])
  def k(aas_hbm, aad_hbm, m_hbm, src_hbm, dst_hbm, ex_hbm, den_hbm,
        si, di, ab, db, eb, mb, zb, den_sh, sem):
    cid = lax.axis_index("c")
    sid = lax.axis_index("s")
    wid = cid * NS + sid

    @pl.loop(0, ROWS_A)
    def _(z):
      zb[z, :] = jnp.zeros((16,), jnp.float32)

    row0 = sid * ROWS_A

    @pl.when(sid < NS - 1)
    def _():
      pltpu.sync_copy(zb, den_sh.at[pl.ds(row0, ROWS_A)])

    @pl.when(sid == NS - 1)
    def _():
      pltpu.sync_copy(zb.at[pl.ds(0, ROWS_LAST)],
                      den_sh.at[pl.ds(row0, ROWS_LAST)])

    pltpu.sync_copy(m_hbm, mb)
    plsc.subcore_barrier()

    mraw = mb[...]
    mv = jnp.where(mraw > 0, mraw, 0.2 * mraw)
    base = wid * E_PER_W

    @pl.loop(0, NCH)
    def _(ci):
      off = base + ci * CH
      pltpu.sync_copy(src_hbm.at[pl.ds(off, CH)], si)
      pltpu.sync_copy(dst_hbm.at[pl.ds(off, CH)], di)
      pltpu.async_copy(aas_hbm.at[si], ab, sem).wait()
      pltpu.async_copy(aad_hbm.at[di], db, sem).wait()

      @pl.loop(0, CH)
      def _(c):
        v = ab[c, :] + db[c, :]
        v = jnp.where(v > 0, v, 0.2 * v)
        eb[c, :] = jnp.exp(v - mv)

      pltpu.sync_copy(eb, ex_hbm.at[pl.ds(off, CH)])
      pltpu.sync_copy(eb, den_sh.at[di], add=True)

    plsc.subcore_barrier()

    @pl.when(sid < NS - 1)
    def _():
      pltpu.sync_copy(den_sh.at[pl.ds(row0, ROWS_A)],
                      den_hbm.at[pl.ds(row0, ROWS_A), pl.ds(cid * 16, 16)])

    @pl.when(sid == NS - 1)
    def _():
      pltpu.sync_copy(den_sh.at[pl.ds(row0, ROWS_LAST)],
                      den_hbm.at[pl.ds(row0, ROWS_LAST), pl.ds(cid * 16, 16)])

  return k(aas, aad, m_raw, src, dst)


def _sc_pass2(h, ex, den, src, dst):
  @functools.partial(
      pl.kernel,
      out_type=jax.ShapeDtypeStruct((2 * N, HID), jnp.float32),
      mesh=_sc_mesh(),
      scratch_types=[
          pltpu.VMEM((CH,), jnp.int32),
          pltpu.VMEM((CH,), jnp.int32),
          pltpu.VMEM((CH, 16), jnp.float32),
          pltpu.VMEM((CH, 32), jnp.float32),
          pltpu.VMEM((CH, HEADS * HID), jnp.float32),
          pltpu.VMEM((CH, HID), jnp.float32),
          pltpu.VMEM((16,), jnp.float32),
          pltpu.VMEM((CH, HID), jnp.float32),
          pltpu.VMEM_SHARED((N, HID), jnp.float32),
          pltpu.SemaphoreType.DMA,
      ])
  def k(h_hbm, ex_hbm, den_hbm, src_hbm, dst_hbm, agg_hbm,
        si, di, eb, ddb, hb, mbuf, wb, zb, agg_sh, sem):
    cid = lax.axis_index("c")
    sid = lax.axis_index("s")
    wid = cid * NS + sid

    @pl.loop(0, CH)
    def _(z):
      for kk in range(HID // 16):
        zb[z, pl.ds(kk * 16, 16)] = jnp.zeros((16,), jnp.float32)

    row0 = sid * ROWS_A

    @pl.when(sid < NS - 1)
    def _():
      for kk in range(7):
        pltpu.sync_copy(zb, agg_sh.at[pl.ds(row0 + kk * CH, CH)])
      pltpu.sync_copy(zb.at[pl.ds(0, 72)],
                      agg_sh.at[pl.ds(row0 + 7 * CH, 72)])

    @pl.when(sid == NS - 1)
    def _():
      for kk in range(6):
        pltpu.sync_copy(zb, agg_sh.at[pl.ds(row0 + kk * CH, CH)])
      pltpu.sync_copy(zb.at[pl.ds(0, 40)],
                      agg_sh.at[pl.ds(row0 + 6 * CH, 40)])

    plsc.subcore_barrier()

    base = wid * E_PER_W

    @pl.loop(0, NCH)
    def _(ci):
      off = base + ci * CH
      pltpu.sync_copy(src_hbm.at[pl.ds(off, CH)], si)
      pltpu.sync_copy(dst_hbm.at[pl.ds(off, CH)], di)
      pltpu.sync_copy(ex_hbm.at[pl.ds(off, CH)], eb)
      pltpu.async_copy(den_hbm.at[di], ddb, sem).wait()
      pltpu.async_copy(h_hbm.at[si], hb, sem).wait()

      @pl.loop(0, CH)
      def _(c):
        dsum = ddb[c, pl.ds(0, 16)] + ddb[c, pl.ds(16, 16)]
        wb[...] = eb[c, :] / (dsum + 1e-16)
        for kk in range(HID // 16):
          acc = wb[0] * hb[c, pl.ds(kk * 16, 16)]
          for hh in range(1, HEADS):
            acc = acc + wb[hh] * hb[c, pl.ds(hh * HID + kk * 16, 16)]
          mbuf[c, pl.ds(kk * 16, 16)] = acc

      pltpu.sync_copy(mbuf, agg_sh.at[di], add=True)

    plsc.subcore_barrier()

    @pl.when(sid < NS - 1)
    def _():
      pltpu.sync_copy(agg_sh.at[pl.ds(row0, ROWS_A)],
                      agg_hbm.at[pl.ds(cid * N + row0, ROWS_A)])

    @pl.when(sid == NS - 1)
    def _():
      pltpu.sync_copy(agg_sh.at[pl.ds(row0, ROWS_LAST)],
                      agg_hbm.at[pl.ds(cid * N + row0, ROWS_LAST)])

  return k(h, ex, den, src, dst)


def _block_diag(att):
  flat = att.reshape(-1)
  r = jnp.arange(HEADS * HID)
  B = jnp.zeros((HEADS * HID, 16), jnp.float32)
  return B.at[r, r // HID].set(flat)


def _gat_layer(xin, W, att_src, att_dst, src, dst, skip_W=None, skip_b=None):
  Bs = _block_diag(att_src)
  Bd = _block_diag(att_dst)
  if skip_W is not None:
    h, aas, aad, ms, md, skip = _dense_block(xin, W, Bs, Bd, skip_W, skip_b)
  else:
    h, aas, aad, ms, md = _dense_block(xin, W, Bs, Bd)
    skip = None
  m_raw = (ms + md).reshape(16)
  ex, den = _sc_pass1(aas, aad, m_raw, src, dst)
  agg = _sc_pass2(h, ex, den, src, dst)
  return agg[:N], agg[N:], skip


def kernel(x, edge_index, batch_idx, W1, att_src1, att_dst1, bias1, bn1_g,
           bn1_b, skip_W, skip_b, W2, att_src2, att_dst2, bias2, bn2_g,
           bn2_b):
  src = edge_index[0].astype(jnp.int32)
  dst = edge_index[1].astype(jnp.int32)

  p0, p1, skip = _gat_layer(x, W1, att_src1, att_dst1, src, dst,
                            skip_W, skip_b)
  hmid = _post_bn_gelu(p0, p1, skip, bias1, bn1_g, bn1_b)

  q0, q1, _ = _gat_layer(hmid, W2, att_src2, att_dst2, src, dst)
  batch_row = batch_idx.astype(jnp.int32).reshape(1, N)
  return _post_bn_gelu_pool(q0, q1, batch_row, bias2, bn2_g, bn2_b)


# speedup vs baseline: 14.2634x; 14.2634x over previous
import functools

import jax
import jax.numpy as jnp
import numpy as np
from jax import lax
from jax.experimental import pallas as pl
from jax.experimental.pallas import tpu as pltpu
from jax.experimental.pallas import tpu_sc as plsc

N = 10000
E = 320000
D = 128
HID = 128
HEADS = 8
NG = 64

NC = 2
NS = 16
NW = NC * NS
E_PER_W = E // NW
CH = 80
NCH = E_PER_W // CH
CH2 = 40
NCH2 = E_PER_W // CH2
ROWS_A = 632
ROWS_LAST = N - 15 * ROWS_A

BN_BLK = 1000
GRID_N = N // BN_BLK


def _dense_block(x, W, Bs, Bd, skip_W=None, skip_b=None):
  D_in = x.shape[1]
  with_skip = skip_W is not None

  def body(*refs):
    if with_skip:
      (x_ref, w_ref, bs_ref, bd_ref, sw_ref, sb_ref,
       h_ref, as_ref, ad_ref, ms_ref, md_ref, sk_ref) = refs
    else:
      (x_ref, w_ref, bs_ref, bd_ref,
       h_ref, as_ref, ad_ref, ms_ref, md_ref) = refs
    i = pl.program_id(0)
    xb = x_ref[...]
    hb = jnp.dot(xb, w_ref[...], preferred_element_type=jnp.float32)
    h_ref[...] = hb
    a_s = jnp.dot(hb, bs_ref[...], preferred_element_type=jnp.float32)
    a_d = jnp.dot(hb, bd_ref[...], preferred_element_type=jnp.float32)
    as_ref[...] = a_s
    ad_ref[...] = a_d

    @pl.when(i == 0)
    def _():
      ms_ref[...] = jnp.full((1, 16), -1e30, jnp.float32)
      md_ref[...] = jnp.full((1, 16), -1e30, jnp.float32)

    ms_ref[...] = jnp.maximum(ms_ref[...], jnp.max(a_s, axis=0, keepdims=True))
    md_ref[...] = jnp.maximum(md_ref[...], jnp.max(a_d, axis=0, keepdims=True))
    if with_skip:
      sk_ref[...] = (jnp.dot(xb, sw_ref[...], preferred_element_type=jnp.float32)
                     + sb_ref[...])

  out_shape = [
      jax.ShapeDtypeStruct((N, HEADS * HID), jnp.float32),
      jax.ShapeDtypeStruct((N, 16), jnp.float32),
      jax.ShapeDtypeStruct((N, 16), jnp.float32),
      jax.ShapeDtypeStruct((1, 16), jnp.float32),
      jax.ShapeDtypeStruct((1, 16), jnp.float32),
  ]
  out_specs = [
      pl.BlockSpec((BN_BLK, HEADS * HID), lambda i: (i, 0)),
      pl.BlockSpec((BN_BLK, 16), lambda i: (i, 0)),
      pl.BlockSpec((BN_BLK, 16), lambda i: (i, 0)),
      pl.BlockSpec((1, 16), lambda i: (0, 0)),
      pl.BlockSpec((1, 16), lambda i: (0, 0)),
  ]
  in_specs = [
      pl.BlockSpec((BN_BLK, D_in), lambda i: (i, 0)),
      pl.BlockSpec((D_in, HEADS * HID), lambda i: (0, 0)),
      pl.BlockSpec((HEADS * HID, 16), lambda i: (0, 0)),
      pl.BlockSpec((HEADS * HID, 16), lambda i: (0, 0)),
  ]
  args = [x, W, Bs, Bd]
  if with_skip:
    in_specs += [pl.BlockSpec((D_in, HID), lambda i: (0, 0)),
                 pl.BlockSpec((1, HID), lambda i: (0, 0))]
    args += [skip_W, skip_b.reshape(1, HID)]
    out_shape.append(jax.ShapeDtypeStruct((N, HID), jnp.float32))
    out_specs.append(pl.BlockSpec((BN_BLK, HID), lambda i: (i, 0)))

  return pl.pallas_call(
      body, grid=(GRID_N,), in_specs=in_specs, out_specs=out_specs,
      out_shape=out_shape)(*args)


def _erf(x):
  a1, a2, a3, a4, a5 = (0.254829592, -0.284496736, 1.421413741,
                        -1.453152027, 1.061405429)
  p = 0.3275911
  s = jnp.sign(x)
  ax = jnp.abs(x)
  t = 1.0 / (1.0 + p * ax)
  poly = ((((a5 * t + a4) * t + a3) * t + a2) * t + a1) * t
  y = 1.0 - poly * jnp.exp(-ax * ax)
  return s * y


def _gelu(x):
  return x * 0.5 * (1.0 + _erf(x * np.float32(1.0 / np.sqrt(2.0))))


def _post_bn_gelu(p0, p1, skip, bias, g, b):
  def body(p0_ref, p1_ref, sk_ref, bias_ref, g_ref, b_ref, out_ref,
           sums, sumsq):
    ph = pl.program_id(0)
    i = pl.program_id(1)
    gv = (p0_ref[...] + p1_ref[...]) * np.float32(1.0 / HEADS) + bias_ref[...]

    @pl.when(ph == 0)
    def _():
      @pl.when(i == 0)
      def _():
        sums[...] = jnp.zeros((1, HID), jnp.float32)
        sumsq[...] = jnp.zeros((1, HID), jnp.float32)
      sums[...] += jnp.sum(gv, axis=0, keepdims=True)
      sumsq[...] += jnp.sum(gv * gv, axis=0, keepdims=True)

    @pl.when(ph == 1)
    def _():
      mu = sums[...] * np.float32(1.0 / N)
      var = sumsq[...] * np.float32(1.0 / N) - mu * mu
      xn = (gv - mu) * lax.rsqrt(var + 1e-5) * g_ref[...] + b_ref[...]
      out_ref[...] = _gelu(xn + sk_ref[...])

  return pl.pallas_call(
      body, grid=(2, GRID_N),
      in_specs=[
          pl.BlockSpec((BN_BLK, HID), lambda p, i: (i, 0)),
          pl.BlockSpec((BN_BLK, HID), lambda p, i: (i, 0)),
          pl.BlockSpec((BN_BLK, HID), lambda p, i: (i, 0)),
          pl.BlockSpec((1, HID), lambda p, i: (0, 0)),
          pl.BlockSpec((1, HID), lambda p, i: (0, 0)),
          pl.BlockSpec((1, HID), lambda p, i: (0, 0)),
      ],
      out_specs=pl.BlockSpec((BN_BLK, HID), lambda p, i: (i, 0)),
      out_shape=jax.ShapeDtypeStruct((N, HID), jnp.float32),
      scratch_shapes=[pltpu.VMEM((1, HID), jnp.float32),
                      pltpu.VMEM((1, HID), jnp.float32)],
  )(p0, p1, skip, bias.reshape(1, HID), g.reshape(1, HID), b.reshape(1, HID))


def _post_bn_gelu_pool(p0, p1, batch_row, bias, g, b):
  def body(p0_ref, p1_ref, bi_ref, bias_ref, g_ref, b_ref, out_ref,
           sums, sumsq, pool, cnt):
    ph = pl.program_id(0)
    i = pl.program_id(1)
    gv = (p0_ref[...] + p1_ref[...]) * np.float32(1.0 / HEADS) + bias_ref[...]

    @pl.when(ph == 0)
    def _():
      @pl.when(i == 0)
      def _():
        sums[...] = jnp.zeros((1, HID), jnp.float32)
        sumsq[...] = jnp.zeros((1, HID), jnp.float32)
      sums[...] += jnp.sum(gv, axis=0, keepdims=True)
      sumsq[...] += jnp.sum(gv * gv, axis=0, keepdims=True)

    @pl.when(ph == 1)
    def _():
      mu = sums[...] * np.float32(1.0 / N)
      var = sumsq[...] * np.float32(1.0 / N) - mu * mu
      xn = (gv - mu) * lax.rsqrt(var + 1e-5) * g_ref[...] + b_ref[...] + gv
      o = _gelu(xn)
      ids = lax.broadcasted_iota(jnp.int32, (NG, BN_BLK), 0)
      oh = (ids == bi_ref[...].reshape(1, BN_BLK)).astype(jnp.float32)

      @pl.when(i == 0)
      def _():
        pool[...] = jnp.zeros((NG, HID), jnp.float32)
        cnt[...] = jnp.zeros((NG, HID), jnp.float32)

      pool[...] += lax.dot_general(oh, o, (((1,), (0,)), ((), ())),
                                   preferred_element_type=jnp.float32)
      cnt[...] += jnp.broadcast_to(jnp.sum(oh, axis=1, keepdims=True),
                                   (NG, HID))

      @pl.when(i == GRID_N - 1)
      def _():
        out_ref[...] = pool[...] / jnp.maximum(cnt[...], 1.0)

  return pl.pallas_call(
      body, grid=(2, GRID_N),
      in_specs=[
          pl.BlockSpec((BN_BLK, HID), lambda p, i: (i, 0)),
          pl.BlockSpec((BN_BLK, HID), lambda p, i: (i, 0)),
          pl.BlockSpec((1, 1, BN_BLK), lambda p, i: (i, 0, 0)),
          pl.BlockSpec((1, HID), lambda p, i: (0, 0)),
          pl.BlockSpec((1, HID), lambda p, i: (0, 0)),
          pl.BlockSpec((1, HID), lambda p, i: (0, 0)),
      ],
      out_specs=pl.BlockSpec((NG, HID), lambda p, i: (0, 0)),
      out_shape=jax.ShapeDtypeStruct((NG, HID), jnp.float32),
      scratch_shapes=[pltpu.VMEM((1, HID), jnp.float32),
                      pltpu.VMEM((1, HID), jnp.float32),
                      pltpu.VMEM((NG, HID), jnp.float32),
                      pltpu.VMEM((NG, HID), jnp.float32)],
  )(p0, p1, batch_row, bias.reshape(1, HID), g.reshape(1, HID),
    b.reshape(1, HID))


def _sc_mesh():
  return plsc.VectorSubcoreMesh(core_axis_name="c", subcore_axis_name="s")


_SC_PARAMS = pltpu.CompilerParams(use_tc_tiling_on_sc=False)


def _sc_pass1(aas, aad, m_raw, src, dst):
  @functools.partial(
      pl.kernel,
      out_type=(jax.ShapeDtypeStruct((E, 16), jnp.float32),
                jax.ShapeDtypeStruct((2 * N, 16), jnp.float32)),
      mesh=_sc_mesh(),
      compiler_params=_SC_PARAMS,
      scratch_types=[
          pltpu.VMEM((CH,), jnp.int32),
          pltpu.VMEM((CH,), jnp.int32),
          pltpu.VMEM((CH, 16), jnp.float32),
          pltpu.VMEM((CH, 16), jnp.float32),
          pltpu.VMEM((CH, 16), jnp.float32),
          pltpu.VMEM((16,), jnp.float32),
          pltpu.VMEM((ROWS_A, 16), jnp.float32),
          pltpu.VMEM_SHARED((N, 16), jnp.float32),
          pltpu.SemaphoreType.DMA,
      ])
  def k(aas_hbm, aad_hbm, m_hbm, src_hbm, dst_hbm, ex_hbm, den_hbm,
        si, di, ab, db, eb, mb, zb, den_sh, sem):
    cid = lax.axis_index("c")
    sid = lax.axis_index("s")
    wid = cid * NS + sid

    @pl.loop(0, ROWS_A)
    def _(z):
      zb[z, :] = jnp.zeros((16,), jnp.float32)

    row0 = sid * ROWS_A

    @pl.when(sid < NS - 1)
    def _():
      pltpu.sync_copy(zb, den_sh.at[pl.ds(row0, ROWS_A)])

    @pl.when(sid == NS - 1)
    def _():
      pltpu.sync_copy(zb.at[pl.ds(0, ROWS_LAST)],
                      den_sh.at[pl.ds(row0, ROWS_LAST)])

    pltpu.sync_copy(m_hbm, mb)
    plsc.subcore_barrier()

    mraw = mb[...]
    mv = jnp.where(mraw > 0, mraw, 0.2 * mraw)
    base = wid * E_PER_W

    @pl.loop(0, NCH)
    def _(ci):
      off = base + ci * CH
      pltpu.sync_copy(src_hbm.at[pl.ds(off, CH)], si)
      pltpu.sync_copy(dst_hbm.at[pl.ds(off, CH)], di)
      pltpu.async_copy(aas_hbm.at[si], ab, sem).wait()
      pltpu.async_copy(aad_hbm.at[di], db, sem).wait()

      @pl.loop(0, CH)
      def _(c):
        v = ab[c, :] + db[c, :]
        v = jnp.where(v > 0, v, 0.2 * v)
        eb[c, :] = jnp.exp(v - mv)

      pltpu.sync_copy(eb, ex_hbm.at[pl.ds(off, CH)])
      pltpu.sync_copy(eb, den_sh.at[di], add=True)

    plsc.subcore_barrier()

    @pl.when(sid < NS - 1)
    def _():
      pltpu.sync_copy(den_sh.at[pl.ds(row0, ROWS_A)],
                      den_hbm.at[pl.ds(cid * N + row0, ROWS_A)])

    @pl.when(sid == NS - 1)
    def _():
      pltpu.sync_copy(den_sh.at[pl.ds(row0, ROWS_LAST)],
                      den_hbm.at[pl.ds(cid * N + row0, ROWS_LAST)])

  return k(aas, aad, m_raw, src, dst)


def _sc_pass2(h, ex, den, src, dst):
  @functools.partial(
      pl.kernel,
      out_type=jax.ShapeDtypeStruct((2 * N, HID), jnp.float32),
      mesh=_sc_mesh(),
      compiler_params=_SC_PARAMS,
      scratch_types=[
          pltpu.VMEM((CH2,), jnp.int32),
          pltpu.VMEM((CH2,), jnp.int32),
          pltpu.VMEM((CH2,), jnp.int32),
          pltpu.VMEM((CH2, 16), jnp.float32),
          pltpu.VMEM((CH2, 16), jnp.float32),
          pltpu.VMEM((CH2, 16), jnp.float32),
          pltpu.VMEM((CH2, HEADS * HID), jnp.float32),
          pltpu.VMEM((CH2, HID), jnp.float32),
          pltpu.VMEM_SHARED((N, HID), jnp.float32),
          pltpu.SemaphoreType.DMA,
      ])
  def k(h_hbm, ex_hbm, den_hbm, src_hbm, dst_hbm, agg_hbm,
        si, di, di2, eb, d0b, d1b, hb, mbuf, agg_sh, sem):
    cid = lax.axis_index("c")
    sid = lax.axis_index("s")
    wid = cid * NS + sid

    @pl.loop(0, CH2)
    def _(z):
      for kk in range(HID // 16):
        mbuf[z, pl.ds(kk * 16, 16)] = jnp.zeros((16,), jnp.float32)

    row0 = sid * ROWS_A

    @pl.when(sid < NS - 1)
    def _():
      for kk in range(15):
        pltpu.sync_copy(mbuf, agg_sh.at[pl.ds(row0 + kk * CH2, CH2)])
      pltpu.sync_copy(mbuf.at[pl.ds(0, 32)],
                      agg_sh.at[pl.ds(row0 + 15 * CH2, 32)])

    @pl.when(sid == NS - 1)
    def _():
      for kk in range(13):
        pltpu.sync_copy(mbuf, agg_sh.at[pl.ds(row0 + kk * CH2, CH2)])

    plsc.subcore_barrier()

    base = wid * E_PER_W

    @pl.loop(0, NCH2)
    def _(ci):
      off = base + ci * CH2
      pltpu.sync_copy(src_hbm.at[pl.ds(off, CH2)], si)
      pltpu.sync_copy(dst_hbm.at[pl.ds(off, CH2)], di)
      pltpu.sync_copy(ex_hbm.at[pl.ds(off, CH2)], eb)

      for o in (0, 16, CH2 - 16):
        di2[pl.ds(o, 16)] = di[pl.ds(o, 16)] + N

      pltpu.async_copy(den_hbm.at[di], d0b, sem).wait()
      pltpu.async_copy(den_hbm.at[di2], d1b, sem).wait()
      pltpu.async_copy(h_hbm.at[si], hb, sem).wait()

      @pl.loop(0, CH2)
      def _(c):
        dsum = d0b[c, :] + d1b[c, :]
        wv = eb[c, :] / (dsum + 1e-16)
        for kk in range(HID // 16):
          acc = wv[0] * hb[c, pl.ds(kk * 16, 16)]
          for hh in range(1, HEADS):
            acc = acc + wv[hh] * hb[c, pl.ds(hh * HID + kk * 16, 16)]
          mbuf[c, pl.ds(kk * 16, 16)] = acc

      pltpu.sync_copy(mbuf, agg_sh.at[di], add=True)

    plsc.subcore_barrier()

    @pl.when(sid < NS - 1)
    def _():
      pltpu.sync_copy(agg_sh.at[pl.ds(row0, ROWS_A)],
                      agg_hbm.at[pl.ds(cid * N + row0, ROWS_A)])

    @pl.when(sid == NS - 1)
    def _():
      pltpu.sync_copy(agg_sh.at[pl.ds(row0, ROWS_LAST)],
                      agg_hbm.at[pl.ds(cid * N + row0, ROWS_LAST)])

  return k(h, ex, den, src, dst)


def _block_diag(att):
  flat = att.reshape(-1)
  r = jnp.arange(HEADS * HID)
  B = jnp.zeros((HEADS * HID, 16), jnp.float32)
  return B.at[r, r // HID].set(flat)


def _gat_layer(xin, W, att_src, att_dst, src, dst, skip_W=None, skip_b=None):
  Bs = _block_diag(att_src)
  Bd = _block_diag(att_dst)
  if skip_W is not None:
    h, aas, aad, ms, md, skip = _dense_block(xin, W, Bs, Bd, skip_W, skip_b)
  else:
    h, aas, aad, ms, md = _dense_block(xin, W, Bs, Bd)
    skip = None
  m_raw = (ms + md).reshape(16)
  ex, den = _sc_pass1(aas, aad, m_raw, src, dst)
  agg = _sc_pass2(h, ex, den, src, dst)
  return agg[:N], agg[N:], skip


def kernel(x, edge_index, batch_idx, W1, att_src1, att_dst1, bias1, bn1_g,
           bn1_b, skip_W, skip_b, W2, att_src2, att_dst2, bias2, bn2_g,
           bn2_b):
  src = edge_index[0].astype(jnp.int32)
  dst = edge_index[1].astype(jnp.int32)

  p0, p1, skip = _gat_layer(x, W1, att_src1, att_dst1, src, dst,
                            skip_W, skip_b)
  hmid = _post_bn_gelu(p0, p1, skip, bias1, bn1_g, bn1_b)

  q0, q1, _ = _gat_layer(hmid, W2, att_src2, att_dst2, src, dst)
  batch_row = batch_idx.astype(jnp.int32).reshape(GRID_N, 1, BN_BLK)
  return _post_bn_gelu_pool(q0, q1, batch_row, bias2, bn2_g, bn2_b)


# speedup vs baseline: 21.8418x; 1.5313x over previous
import functools

import jax
import jax.numpy as jnp
import numpy as np
from jax import lax
from jax.experimental import pallas as pl
from jax.experimental.pallas import tpu as pltpu
from jax.experimental.pallas import tpu_sc as plsc

N = 10000
E = 320000
D = 128
HID = 128
HEADS = 8
NG = 64

NC = 2
NS = 16
NW = NC * NS
E_PER_W = E // NW
CH = 80
NCH = E_PER_W // CH

CHK = 16
SUPC = 40
NSUP = 16
EPW = CHK * SUPC * NSUP
EP = EPW * NW
PAD_W = EPW - E_PER_W
ROWS_A = 632
ROWS_LAST = N - 15 * ROWS_A

BN_BLK = 1000
GRID_N = N // BN_BLK


def _dense_block(x, W, Bs, Bd, skip_W=None, skip_b=None):
  D_in = x.shape[1]
  with_skip = skip_W is not None

  def body(*refs):
    if with_skip:
      (x_ref, w_ref, bs_ref, bd_ref, sw_ref, sb_ref,
       h_ref, as_ref, ad_ref, ms_ref, md_ref, sk_ref) = refs
    else:
      (x_ref, w_ref, bs_ref, bd_ref,
       h_ref, as_ref, ad_ref, ms_ref, md_ref) = refs
    i = pl.program_id(0)
    xb = x_ref[...]
    hb = jnp.dot(xb, w_ref[...], preferred_element_type=jnp.float32)
    h_ref[...] = hb
    a_s = jnp.dot(hb, bs_ref[...], preferred_element_type=jnp.float32)
    a_d = jnp.dot(hb, bd_ref[...], preferred_element_type=jnp.float32)
    as_ref[...] = a_s
    ad_ref[...] = a_d

    @pl.when(i == 0)
    def _():
      ms_ref[...] = jnp.full((1, 16), -1e30, jnp.float32)
      md_ref[...] = jnp.full((1, 16), -1e30, jnp.float32)

    ms_ref[...] = jnp.maximum(ms_ref[...], jnp.max(a_s, axis=0, keepdims=True))
    md_ref[...] = jnp.maximum(md_ref[...], jnp.max(a_d, axis=0, keepdims=True))
    if with_skip:
      sk_ref[...] = (jnp.dot(xb, sw_ref[...], preferred_element_type=jnp.float32)
                     + sb_ref[...])

  out_shape = [
      jax.ShapeDtypeStruct((N, HEADS * HID), jnp.float32),
      jax.ShapeDtypeStruct((N, 16), jnp.float32),
      jax.ShapeDtypeStruct((N, 16), jnp.float32),
      jax.ShapeDtypeStruct((1, 16), jnp.float32),
      jax.ShapeDtypeStruct((1, 16), jnp.float32),
  ]
  out_specs = [
      pl.BlockSpec((BN_BLK, HEADS * HID), lambda i: (i, 0)),
      pl.BlockSpec((BN_BLK, 16), lambda i: (i, 0)),
      pl.BlockSpec((BN_BLK, 16), lambda i: (i, 0)),
      pl.BlockSpec((1, 16), lambda i: (0, 0)),
      pl.BlockSpec((1, 16), lambda i: (0, 0)),
  ]
  in_specs = [
      pl.BlockSpec((BN_BLK, D_in), lambda i: (i, 0)),
      pl.BlockSpec((D_in, HEADS * HID), lambda i: (0, 0)),
      pl.BlockSpec((HEADS * HID, 16), lambda i: (0, 0)),
      pl.BlockSpec((HEADS * HID, 16), lambda i: (0, 0)),
  ]
  args = [x, W, Bs, Bd]
  if with_skip:
    in_specs += [pl.BlockSpec((D_in, HID), lambda i: (0, 0)),
                 pl.BlockSpec((1, HID), lambda i: (0, 0))]
    args += [skip_W, skip_b.reshape(1, HID)]
    out_shape.append(jax.ShapeDtypeStruct((N, HID), jnp.float32))
    out_specs.append(pl.BlockSpec((BN_BLK, HID), lambda i: (i, 0)))

  return pl.pallas_call(
      body, grid=(GRID_N,), in_specs=in_specs, out_specs=out_specs,
      out_shape=out_shape)(*args)


def _erf(x):
  a1, a2, a3, a4, a5 = (0.254829592, -0.284496736, 1.421413741,
                        -1.453152027, 1.061405429)
  p = 0.3275911
  s = jnp.sign(x)
  ax = jnp.abs(x)
  t = 1.0 / (1.0 + p * ax)
  poly = ((((a5 * t + a4) * t + a3) * t + a2) * t + a1) * t
  y = 1.0 - poly * jnp.exp(-ax * ax)
  return s * y


def _gelu(x):
  return x * 0.5 * (1.0 + _erf(x * np.float32(1.0 / np.sqrt(2.0))))


def _post_bn_gelu(p0, p1, skip, bias, g, b):
  def body(p0_ref, p1_ref, sk_ref, bias_ref, g_ref, b_ref, out_ref,
           sums, sumsq):
    ph = pl.program_id(0)
    i = pl.program_id(1)
    gv = (p0_ref[...] + p1_ref[...]) * np.float32(1.0 / HEADS) + bias_ref[...]

    @pl.when(ph == 0)
    def _():
      @pl.when(i == 0)
      def _():
        sums[...] = jnp.zeros((1, HID), jnp.float32)
        sumsq[...] = jnp.zeros((1, HID), jnp.float32)
      sums[...] += jnp.sum(gv, axis=0, keepdims=True)
      sumsq[...] += jnp.sum(gv * gv, axis=0, keepdims=True)

    @pl.when(ph == 1)
    def _():
      mu = sums[...] * np.float32(1.0 / N)
      var = sumsq[...] * np.float32(1.0 / N) - mu * mu
      xn = (gv - mu) * lax.rsqrt(var + 1e-5) * g_ref[...] + b_ref[...]
      out_ref[...] = _gelu(xn + sk_ref[...])

  return pl.pallas_call(
      body, grid=(2, GRID_N),
      in_specs=[
          pl.BlockSpec((BN_BLK, HID), lambda p, i: (i, 0)),
          pl.BlockSpec((BN_BLK, HID), lambda p, i: (i, 0)),
          pl.BlockSpec((BN_BLK, HID), lambda p, i: (i, 0)),
          pl.BlockSpec((1, HID), lambda p, i: (0, 0)),
          pl.BlockSpec((1, HID), lambda p, i: (0, 0)),
          pl.BlockSpec((1, HID), lambda p, i: (0, 0)),
      ],
      out_specs=pl.BlockSpec((BN_BLK, HID), lambda p, i: (i, 0)),
      out_shape=jax.ShapeDtypeStruct((N, HID), jnp.float32),
      scratch_shapes=[pltpu.VMEM((1, HID), jnp.float32),
                      pltpu.VMEM((1, HID), jnp.float32)],
  )(p0, p1, skip, bias.reshape(1, HID), g.reshape(1, HID), b.reshape(1, HID))


def _post_bn_gelu_pool(p0, p1, batch_row, bias, g, b):
  def body(p0_ref, p1_ref, bi_ref, bias_ref, g_ref, b_ref, out_ref,
           sums, sumsq, pool, cnt):
    ph = pl.program_id(0)
    i = pl.program_id(1)
    gv = (p0_ref[...] + p1_ref[...]) * np.float32(1.0 / HEADS) + bias_ref[...]

    @pl.when(ph == 0)
    def _():
      @pl.when(i == 0)
      def _():
        sums[...] = jnp.zeros((1, HID), jnp.float32)
        sumsq[...] = jnp.zeros((1, HID), jnp.float32)
      sums[...] += jnp.sum(gv, axis=0, keepdims=True)
      sumsq[...] += jnp.sum(gv * gv, axis=0, keepdims=True)

    @pl.when(ph == 1)
    def _():
      mu = sums[...] * np.float32(1.0 / N)
      var = sumsq[...] * np.float32(1.0 / N) - mu * mu
      xn = (gv - mu) * lax.rsqrt(var + 1e-5) * g_ref[...] + b_ref[...] + gv
      o = _gelu(xn)
      ids = lax.broadcasted_iota(jnp.int32, (NG, BN_BLK), 0)
      oh = (ids == bi_ref[...].reshape(1, BN_BLK)).astype(jnp.float32)

      @pl.when(i == 0)
      def _():
        pool[...] = jnp.zeros((NG, HID), jnp.float32)
        cnt[...] = jnp.zeros((NG, HID), jnp.float32)

      pool[...] += lax.dot_general(oh, o, (((1,), (0,)), ((), ())),
                                   preferred_element_type=jnp.float32)
      cnt[...] += jnp.broadcast_to(jnp.sum(oh, axis=1, keepdims=True),
                                   (NG, HID))

      @pl.when(i == GRID_N - 1)
      def _():
        out_ref[...] = pool[...] / jnp.maximum(cnt[...], 1.0)

  return pl.pallas_call(
      body, grid=(2, GRID_N),
      in_specs=[
          pl.BlockSpec((BN_BLK, HID), lambda p, i: (i, 0)),
          pl.BlockSpec((BN_BLK, HID), lambda p, i: (i, 0)),
          pl.BlockSpec((1, 1, BN_BLK), lambda p, i: (i, 0, 0)),
          pl.BlockSpec((1, HID), lambda p, i: (0, 0)),
          pl.BlockSpec((1, HID), lambda p, i: (0, 0)),
          pl.BlockSpec((1, HID), lambda p, i: (0, 0)),
      ],
      out_specs=pl.BlockSpec((NG, HID), lambda p, i: (0, 0)),
      out_shape=jax.ShapeDtypeStruct((NG, HID), jnp.float32),
      scratch_shapes=[pltpu.VMEM((1, HID), jnp.float32),
                      pltpu.VMEM((1, HID), jnp.float32),
                      pltpu.VMEM((NG, HID), jnp.float32),
                      pltpu.VMEM((NG, HID), jnp.float32)],
  )(p0, p1, batch_row, bias.reshape(1, HID), g.reshape(1, HID),
    b.reshape(1, HID))


def _sc_mesh():
  return plsc.VectorSubcoreMesh(core_axis_name="c", subcore_axis_name="s")


_SC_PARAMS = pltpu.CompilerParams(use_tc_tiling_on_sc=False)


def _sc_pass1(aas, aad, m_raw, src, dst):
  @functools.partial(
      pl.kernel,
      out_type=(jax.ShapeDtypeStruct((EP, 16), jnp.float32),
                jax.ShapeDtypeStruct((2 * N, 16), jnp.float32)),
      mesh=_sc_mesh(),
      compiler_params=_SC_PARAMS,
      scratch_types=[
          pltpu.VMEM((CH,), jnp.int32),
          pltpu.VMEM((CH,), jnp.int32),
          pltpu.VMEM((CH, 16), jnp.float32),
          pltpu.VMEM((CH, 16), jnp.float32),
          pltpu.VMEM((CH, 16), jnp.float32),
          pltpu.VMEM((16,), jnp.float32),
          pltpu.VMEM((ROWS_A, 16), jnp.float32),
          pltpu.VMEM_SHARED((N, 16), jnp.float32),
          pltpu.SemaphoreType.DMA,
      ])
  def k(aas_hbm, aad_hbm, m_hbm, src_hbm, dst_hbm, ex_hbm, den_hbm,
        si, di, ab, db, eb, mb, zb, den_sh, sem):
    cid = lax.axis_index("c")
    sid = lax.axis_index("s")
    wid = cid * NS + sid

    @pl.loop(0, ROWS_A)
    def _(z):
      zb[z, :] = jnp.zeros((16,), jnp.float32)

    row0 = sid * ROWS_A

    @pl.when(sid < NS - 1)
    def _():
      pltpu.sync_copy(zb, den_sh.at[pl.ds(row0, ROWS_A)])

    @pl.when(sid == NS - 1)
    def _():
      pltpu.sync_copy(zb.at[pl.ds(0, ROWS_LAST)],
                      den_sh.at[pl.ds(row0, ROWS_LAST)])

    pltpu.sync_copy(m_hbm, mb)
    plsc.subcore_barrier()

    mraw = mb[...]
    mv = jnp.where(mraw > 0, mraw, 0.2 * mraw)
    base = wid * E_PER_W
    base_p = wid * EPW

    @pl.loop(0, NCH)
    def _(ci):
      off = base + ci * CH
      pltpu.sync_copy(src_hbm.at[pl.ds(off, CH)], si)
      pltpu.sync_copy(dst_hbm.at[pl.ds(off, CH)], di)
      pltpu.async_copy(aas_hbm.at[si], ab, sem).wait()
      pltpu.async_copy(aad_hbm.at[di], db, sem).wait()

      @pl.loop(0, CH)
      def _(c):
        v = ab[c, :] + db[c, :]
        v = jnp.where(v > 0, v, 0.2 * v)
        eb[c, :] = jnp.exp(v - mv)

      pltpu.sync_copy(eb, ex_hbm.at[pl.ds(base_p + ci * CH, CH)])
      pltpu.sync_copy(eb, den_sh.at[di], add=True)

    pltpu.sync_copy(zb.at[pl.ds(0, PAD_W)],
                    ex_hbm.at[pl.ds(base_p + E_PER_W, PAD_W)])

    plsc.subcore_barrier()

    @pl.when(sid < NS - 1)
    def _():
      pltpu.sync_copy(den_sh.at[pl.ds(row0, ROWS_A)],
                      den_hbm.at[pl.ds(cid * N + row0, ROWS_A)])

    @pl.when(sid == NS - 1)
    def _():
      pltpu.sync_copy(den_sh.at[pl.ds(row0, ROWS_LAST)],
                      den_hbm.at[pl.ds(cid * N + row0, ROWS_LAST)])

  return k(aas, aad, m_raw, src, dst)


def _den_recip(den):
  d0 = den[:N]
  d1 = den[N:]

  def body(a_ref, b_ref, o_ref):
    o_ref[...] = 1.0 / (a_ref[...] + b_ref[...] + 1e-16)

  return pl.pallas_call(
      body, grid=(GRID_N,),
      in_specs=[pl.BlockSpec((BN_BLK, 16), lambda i: (i, 0)),
                pl.BlockSpec((BN_BLK, 16), lambda i: (i, 0))],
      out_specs=pl.BlockSpec((BN_BLK, 16), lambda i: (i, 0)),
      out_shape=jax.ShapeDtypeStruct((N, 16), jnp.float32))(d0, d1)


def _sc_pass2(h, exP, rden, srcP, dstP):
  @functools.partial(
      pl.kernel,
      out_type=jax.ShapeDtypeStruct((2 * N, HID), jnp.float32),
      mesh=_sc_mesh(),
      compiler_params=_SC_PARAMS,
      scratch_types=[
          pltpu.VMEM((SUPC, 16), jnp.int32),
          pltpu.VMEM((SUPC, 16), jnp.int32),
          pltpu.VMEM((SUPC, 16), jnp.int32),
          pltpu.VMEM((SUPC, 16), jnp.int32),
          pltpu.VMEM((CHK, 16), jnp.float32),
          pltpu.VMEM((CHK, 16), jnp.float32),
          pltpu.VMEM((CHK, 16), jnp.float32),
          pltpu.VMEM((CHK, 16), jnp.float32),
          pltpu.VMEM((CHK, HEADS * HID), jnp.float32),
          pltpu.VMEM((CHK, HEADS * HID), jnp.float32),
          pltpu.VMEM((CHK, HID), jnp.float32),
          pltpu.VMEM((CHK, HID), jnp.float32),
          pltpu.VMEM_SHARED((N, HID), jnp.float32),
          pltpu.SemaphoreType.DMA,
          pltpu.SemaphoreType.DMA,
          pltpu.SemaphoreType.DMA,
          pltpu.SemaphoreType.DMA,
          pltpu.SemaphoreType.DMA,
          pltpu.SemaphoreType.DMA,
      ])
  def k(h_hbm, ex_hbm, rd_hbm, src_hbm, dst_hbm, agg_hbm,
        sbuf0, sbuf1, dbuf0, dbuf1, eb0, eb1, rb0, rb1, hb0, hb1,
        mb0, mb1, agg_sh, isem0, isem1, gsem0, gsem1, ssem0, ssem1):
    cid = lax.axis_index("c")
    sid = lax.axis_index("s")
    wid = cid * NS + sid
    sbuf = (sbuf0, sbuf1)
    dbuf = (dbuf0, dbuf1)
    eb = (eb0, eb1)
    rb = (rb0, rb1)
    hb = (hb0, hb1)
    mb = (mb0, mb1)
    isem = (isem0, isem1)
    gsem = (gsem0, gsem1)
    ssem = (ssem0, ssem1)

    @pl.loop(0, CHK)
    def _(z):
      for kk2 in range(HID // 16):
        mb0[z, pl.ds(kk2 * 16, 16)] = jnp.zeros((16,), jnp.float32)

    row0 = sid * ROWS_A

    @pl.when(sid < NS - 1)
    def _():
      @pl.loop(0, 39)
      def _(kk2):
        pltpu.sync_copy(mb0, agg_sh.at[pl.ds(row0 + kk2 * CHK, CHK)])
      pltpu.sync_copy(mb0.at[pl.ds(0, 8)], agg_sh.at[pl.ds(row0 + 624, 8)])

    @pl.when(sid == NS - 1)
    def _():
      @pl.loop(0, 32)
      def _(kk2):
        pltpu.sync_copy(mb0, agg_sh.at[pl.ds(row0 + kk2 * CHK, CHK)])
      pltpu.sync_copy(mb0.at[pl.ds(0, 8)], agg_sh.at[pl.ds(row0 + 512, 8)])

    plsc.subcore_barrier()

    ebase = wid * EPW
    irow0 = wid * (EPW // 16)

    def issue_chunk(off, sb_ref, db_ref, jrow, slot):
      pltpu.async_copy(ex_hbm.at[pl.ds(off, CHK)], eb[slot], gsem[slot])
      pltpu.async_copy(rd_hbm.at[db_ref.at[jrow]], rb[slot], gsem[slot])
      pltpu.async_copy(h_hbm.at[sb_ref.at[jrow]], hb[slot], gsem[slot])

    def wait_chunk(sb_ref, db_ref, slot):
      pltpu.make_async_copy(ex_hbm.at[pl.ds(ebase, CHK)], eb[slot],
                            gsem[slot]).wait()
      pltpu.make_async_copy(rd_hbm.at[db_ref.at[0]], rb[slot],
                            gsem[slot]).wait()
      pltpu.make_async_copy(h_hbm.at[sb_ref.at[0]], hb[slot],
                            gsem[slot]).wait()

    pltpu.sync_copy(src_hbm.at[pl.ds(irow0, SUPC)], sbuf0)
    pltpu.sync_copy(dst_hbm.at[pl.ds(irow0, SUPC)], dbuf0)
    issue_chunk(ebase, sbuf0, dbuf0, 0, 0)

    @pl.loop(0, NSUP // 2)
    def _(tp):
      for ss in (0, 1):
        t = tp * 2 + ss
        tbase = ebase + t * (SUPC * CHK)
        sb = sbuf[ss]
        db = dbuf[ss]

        @pl.loop(0, SUPC // 2)
        def _(jj):
          for b in (0, 1):
            j = jj * 2 + b
            nb = 1 - b
            wait_chunk(sb, db, b)

            if b == 0:
              issue_chunk(tbase + (j + 1) * CHK, sb, db, j + 1, nb)
            else:
              @pl.when(jj < SUPC // 2 - 1)
              def _():
                issue_chunk(tbase + (j + 1) * CHK, sb, db, j + 1, nb)

              @pl.when((jj == SUPC // 2 - 1) & (t < NSUP - 1))
              def _():
                pltpu.make_async_copy(
                    src_hbm.at[pl.ds(irow0, SUPC)], sbuf[1 - ss],
                    isem[1 - ss]).wait()
                pltpu.make_async_copy(
                    dst_hbm.at[pl.ds(irow0, SUPC)], dbuf[1 - ss],
                    isem[1 - ss]).wait()
                issue_chunk(tbase + SUPC * CHK, sbuf[1 - ss], dbuf[1 - ss],
                            0, nb)

            @pl.when((t > 0) | (jj >= 1))
            def _():
              pltpu.make_async_copy(mb[b], agg_sh.at[db.at[j]],
                                    ssem[b]).wait()

            @pl.loop(0, CHK)
            def _(c):
              wv = eb[b][c, :] * rb[b][c, :]
              for kk2 in range(HID // 16):
                acc = wv[0] * hb[b][c, pl.ds(kk2 * 16, 16)]
                for hh in range(1, HEADS):
                  acc = acc + wv[hh] * hb[b][c, pl.ds(hh * HID + kk2 * 16, 16)]
                mb[b][c, pl.ds(kk2 * 16, 16)] = acc

            pltpu.async_copy(mb[b], agg_sh.at[db.at[j]], ssem[b], add=True)

            if b == 1:
              @pl.when((jj == 0) & (t < NSUP - 1))
              def _():
                nrow = irow0 + (t + 1) * SUPC
                pltpu.async_copy(src_hbm.at[pl.ds(nrow, SUPC)],
                                 sbuf[1 - ss], isem[1 - ss])
                pltpu.async_copy(dst_hbm.at[pl.ds(nrow, SUPC)],
                                 dbuf[1 - ss], isem[1 - ss])

    pltpu.make_async_copy(mb0, agg_sh.at[dbuf1.at[0]], ssem0).wait()
    pltpu.make_async_copy(mb1, agg_sh.at[dbuf1.at[0]], ssem1).wait()

    plsc.subcore_barrier()

    @pl.when(sid < NS - 1)
    def _():
      pltpu.sync_copy(agg_sh.at[pl.ds(row0, ROWS_A)],
                      agg_hbm.at[pl.ds(cid * N + row0, ROWS_A)])

    @pl.when(sid == NS - 1)
    def _():
      pltpu.sync_copy(agg_sh.at[pl.ds(row0, ROWS_LAST)],
                      agg_hbm.at[pl.ds(cid * N + row0, ROWS_LAST)])

  return k(h, exP, rden, srcP, dstP)


def _block_diag(att):
  flat = att.reshape(-1)
  r = jnp.arange(HEADS * HID)
  B = jnp.zeros((HEADS * HID, 16), jnp.float32)
  return B.at[r, r // HID].set(flat)


def _gat_layer(xin, W, att_src, att_dst, src, dst, srcP, dstP,
               skip_W=None, skip_b=None):
  Bs = _block_diag(att_src)
  Bd = _block_diag(att_dst)
  if skip_W is not None:
    h, aas, aad, ms, md, skip = _dense_block(xin, W, Bs, Bd, skip_W, skip_b)
  else:
    h, aas, aad, ms, md = _dense_block(xin, W, Bs, Bd)
    skip = None
  m_raw = (ms + md).reshape(16)
  exP, den = _sc_pass1(aas, aad, m_raw, src, dst)
  rden = _den_recip(den)
  agg = _sc_pass2(h, exP, rden, srcP, dstP)
  return agg[:N], agg[N:], skip


def kernel(x, edge_index, batch_idx, W1, att_src1, att_dst1, bias1, bn1_g,
           bn1_b, skip_W, skip_b, W2, att_src2, att_dst2, bias2, bn2_g,
           bn2_b):
  src = edge_index[0].astype(jnp.int32)
  dst = edge_index[1].astype(jnp.int32)
  srcP = jnp.pad(src.reshape(NW, E_PER_W),
                 ((0, 0), (0, PAD_W))).reshape(EP // 16, 16)
  dstP = jnp.pad(dst.reshape(NW, E_PER_W),
                 ((0, 0), (0, PAD_W))).reshape(EP // 16, 16)

  p0, p1, skip = _gat_layer(x, W1, att_src1, att_dst1, src, dst, srcP, dstP,
                            skip_W, skip_b)
  hmid = _post_bn_gelu(p0, p1, skip, bias1, bn1_g, bn1_b)

  q0, q1, _ = _gat_layer(hmid, W2, att_src2, att_dst2, src, dst, srcP, dstP)
  batch_row = batch_idx.astype(jnp.int32).reshape(GRID_N, 1, BN_BLK)
  return _post_bn_gelu_pool(q0, q1, batch_row, bias2, bn2_g, bn2_b)


# speedup vs baseline: 23.9248x; 1.0954x over previous
import functools

import jax
import jax.numpy as jnp
import numpy as np
from jax import lax
from jax.experimental import pallas as pl
from jax.experimental.pallas import tpu as pltpu
from jax.experimental.pallas import tpu_sc as plsc

N = 10000
E = 320000
D = 128
HID = 128
HEADS = 8
NG = 64

NC = 2
NS = 16
NW = NC * NS
E_PER_W = E // NW
CH1 = 40
NCH1 = E_PER_W // CH1
IR1 = E // CH1

CHK = 16
SUPC = 40
NSUP = 16
EPW = CHK * SUPC * NSUP
EP = EPW * NW
PAD_W = EPW - E_PER_W
ROWS_A = 632
ROWS_LAST = N - 15 * ROWS_A

BN_BLK = 1000
GRID_N = N // BN_BLK


def _dense_block(x, W, Bs, Bd, skip_W=None, skip_b=None):
  D_in = x.shape[1]
  with_skip = skip_W is not None

  def body(*refs):
    if with_skip:
      (x_ref, w_ref, bs_ref, bd_ref, sw_ref, sb_ref,
       h_ref, as_ref, ad_ref, ms_ref, md_ref, sk_ref) = refs
    else:
      (x_ref, w_ref, bs_ref, bd_ref,
       h_ref, as_ref, ad_ref, ms_ref, md_ref) = refs
    i = pl.program_id(0)
    xb = x_ref[...]
    hb = jnp.dot(xb, w_ref[...], preferred_element_type=jnp.float32)
    h_ref[...] = hb
    a_s = jnp.dot(hb, bs_ref[...], preferred_element_type=jnp.float32)
    a_d = jnp.dot(hb, bd_ref[...], preferred_element_type=jnp.float32)
    as_ref[...] = a_s
    ad_ref[...] = a_d

    @pl.when(i == 0)
    def _():
      ms_ref[...] = jnp.full((1, 16), -1e30, jnp.float32)
      md_ref[...] = jnp.full((1, 16), -1e30, jnp.float32)

    ms_ref[...] = jnp.maximum(ms_ref[...], jnp.max(a_s, axis=0, keepdims=True))
    md_ref[...] = jnp.maximum(md_ref[...], jnp.max(a_d, axis=0, keepdims=True))
    if with_skip:
      sk_ref[...] = (jnp.dot(xb, sw_ref[...], preferred_element_type=jnp.float32)
                     + sb_ref[...])

  out_shape = [
      jax.ShapeDtypeStruct((N, HEADS * HID), jnp.float32),
      jax.ShapeDtypeStruct((N, 16), jnp.float32),
      jax.ShapeDtypeStruct((N, 16), jnp.float32),
      jax.ShapeDtypeStruct((1, 16), jnp.float32),
      jax.ShapeDtypeStruct((1, 16), jnp.float32),
  ]
  out_specs = [
      pl.BlockSpec((BN_BLK, HEADS * HID), lambda i: (i, 0)),
      pl.BlockSpec((BN_BLK, 16), lambda i: (i, 0)),
      pl.BlockSpec((BN_BLK, 16), lambda i: (i, 0)),
      pl.BlockSpec((1, 16), lambda i: (0, 0)),
      pl.BlockSpec((1, 16), lambda i: (0, 0)),
  ]
  in_specs = [
      pl.BlockSpec((BN_BLK, D_in), lambda i: (i, 0)),
      pl.BlockSpec((D_in, HEADS * HID), lambda i: (0, 0)),
      pl.BlockSpec((HEADS * HID, 16), lambda i: (0, 0)),
      pl.BlockSpec((HEADS * HID, 16), lambda i: (0, 0)),
  ]
  args = [x, W, Bs, Bd]
  if with_skip:
    in_specs += [pl.BlockSpec((D_in, HID), lambda i: (0, 0)),
                 pl.BlockSpec((1, HID), lambda i: (0, 0))]
    args += [skip_W, skip_b.reshape(1, HID)]
    out_shape.append(jax.ShapeDtypeStruct((N, HID), jnp.float32))
    out_specs.append(pl.BlockSpec((BN_BLK, HID), lambda i: (i, 0)))

  return pl.pallas_call(
      body, grid=(GRID_N,), in_specs=in_specs, out_specs=out_specs,
      out_shape=out_shape)(*args)


def _erf(x):
  a1, a2, a3, a4, a5 = (0.254829592, -0.284496736, 1.421413741,
                        -1.453152027, 1.061405429)
  p = 0.3275911
  s = jnp.sign(x)
  ax = jnp.abs(x)
  t = 1.0 / (1.0 + p * ax)
  poly = ((((a5 * t + a4) * t + a3) * t + a2) * t + a1) * t
  y = 1.0 - poly * jnp.exp(-ax * ax)
  return s * y


def _gelu(x):
  return x * 0.5 * (1.0 + _erf(x * np.float32(1.0 / np.sqrt(2.0))))


def _post_bn_gelu(p0, p1, skip, bias, g, b):
  def body(p0_ref, p1_ref, sk_ref, bias_ref, g_ref, b_ref, out_ref,
           sums, sumsq):
    ph = pl.program_id(0)
    i = pl.program_id(1)
    gv = (p0_ref[...] + p1_ref[...]) * np.float32(1.0 / HEADS) + bias_ref[...]

    @pl.when(ph == 0)
    def _():
      @pl.when(i == 0)
      def _():
        sums[...] = jnp.zeros((1, HID), jnp.float32)
        sumsq[...] = jnp.zeros((1, HID), jnp.float32)
      sums[...] += jnp.sum(gv, axis=0, keepdims=True)
      sumsq[...] += jnp.sum(gv * gv, axis=0, keepdims=True)

    @pl.when(ph == 1)
    def _():
      mu = sums[...] * np.float32(1.0 / N)
      var = sumsq[...] * np.float32(1.0 / N) - mu * mu
      xn = (gv - mu) * lax.rsqrt(var + 1e-5) * g_ref[...] + b_ref[...]
      out_ref[...] = _gelu(xn + sk_ref[...])

  return pl.pallas_call(
      body, grid=(2, GRID_N),
      in_specs=[
          pl.BlockSpec((BN_BLK, HID), lambda p, i: (i, 0)),
          pl.BlockSpec((BN_BLK, HID), lambda p, i: (i, 0)),
          pl.BlockSpec((BN_BLK, HID), lambda p, i: (i, 0)),
          pl.BlockSpec((1, HID), lambda p, i: (0, 0)),
          pl.BlockSpec((1, HID), lambda p, i: (0, 0)),
          pl.BlockSpec((1, HID), lambda p, i: (0, 0)),
      ],
      out_specs=pl.BlockSpec((BN_BLK, HID), lambda p, i: (i, 0)),
      out_shape=jax.ShapeDtypeStruct((N, HID), jnp.float32),
      scratch_shapes=[pltpu.VMEM((1, HID), jnp.float32),
                      pltpu.VMEM((1, HID), jnp.float32)],
  )(p0, p1, skip, bias.reshape(1, HID), g.reshape(1, HID), b.reshape(1, HID))


def _post_bn_gelu_pool(p0, p1, batch_row, bias, g, b):
  def body(p0_ref, p1_ref, bi_ref, bias_ref, g_ref, b_ref, out_ref,
           sums, sumsq, pool, cnt):
    ph = pl.program_id(0)
    i = pl.program_id(1)
    gv = (p0_ref[...] + p1_ref[...]) * np.float32(1.0 / HEADS) + bias_ref[...]

    @pl.when(ph == 0)
    def _():
      @pl.when(i == 0)
      def _():
        sums[...] = jnp.zeros((1, HID), jnp.float32)
        sumsq[...] = jnp.zeros((1, HID), jnp.float32)
      sums[...] += jnp.sum(gv, axis=0, keepdims=True)
      sumsq[...] += jnp.sum(gv * gv, axis=0, keepdims=True)

    @pl.when(ph == 1)
    def _():
      mu = sums[...] * np.float32(1.0 / N)
      var = sumsq[...] * np.float32(1.0 / N) - mu * mu
      xn = (gv - mu) * lax.rsqrt(var + 1e-5) * g_ref[...] + b_ref[...] + gv
      o = _gelu(xn)
      ids = lax.broadcasted_iota(jnp.int32, (NG, BN_BLK), 0)
      oh = (ids == bi_ref[...].reshape(1, BN_BLK)).astype(jnp.float32)

      @pl.when(i == 0)
      def _():
        pool[...] = jnp.zeros((NG, HID), jnp.float32)
        cnt[...] = jnp.zeros((NG, HID), jnp.float32)

      pool[...] += lax.dot_general(oh, o, (((1,), (0,)), ((), ())),
                                   preferred_element_type=jnp.float32)
      cnt[...] += jnp.broadcast_to(jnp.sum(oh, axis=1, keepdims=True),
                                   (NG, HID))

      @pl.when(i == GRID_N - 1)
      def _():
        out_ref[...] = pool[...] / jnp.maximum(cnt[...], 1.0)

  return pl.pallas_call(
      body, grid=(2, GRID_N),
      in_specs=[
          pl.BlockSpec((BN_BLK, HID), lambda p, i: (i, 0)),
          pl.BlockSpec((BN_BLK, HID), lambda p, i: (i, 0)),
          pl.BlockSpec((1, 1, BN_BLK), lambda p, i: (i, 0, 0)),
          pl.BlockSpec((1, HID), lambda p, i: (0, 0)),
          pl.BlockSpec((1, HID), lambda p, i: (0, 0)),
          pl.BlockSpec((1, HID), lambda p, i: (0, 0)),
      ],
      out_specs=pl.BlockSpec((NG, HID), lambda p, i: (0, 0)),
      out_shape=jax.ShapeDtypeStruct((NG, HID), jnp.float32),
      scratch_shapes=[pltpu.VMEM((1, HID), jnp.float32),
                      pltpu.VMEM((1, HID), jnp.float32),
                      pltpu.VMEM((NG, HID), jnp.float32),
                      pltpu.VMEM((NG, HID), jnp.float32)],
  )(p0, p1, batch_row, bias.reshape(1, HID), g.reshape(1, HID),
    b.reshape(1, HID))


def _sc_mesh():
  return plsc.VectorSubcoreMesh(core_axis_name="c", subcore_axis_name="s")


_SC_PARAMS = pltpu.CompilerParams(use_tc_tiling_on_sc=False)


def _sc_pass1(aas, aad, m_raw, src40, dst40):
  @functools.partial(
      pl.kernel,
      out_type=(jax.ShapeDtypeStruct((EP, 16), jnp.float32),
                jax.ShapeDtypeStruct((2 * N, 16), jnp.float32)),
      mesh=_sc_mesh(),
      compiler_params=_SC_PARAMS,
      scratch_types=[
          pltpu.VMEM((NCH1, CH1), jnp.int32),
          pltpu.VMEM((NCH1, CH1), jnp.int32),
          pltpu.VMEM((CH1, 16), jnp.float32),
          pltpu.VMEM((CH1, 16), jnp.float32),
          pltpu.VMEM((CH1, 16), jnp.float32),
          pltpu.VMEM((CH1, 16), jnp.float32),
          pltpu.VMEM((CH1, 16), jnp.float32),
          pltpu.VMEM((CH1, 16), jnp.float32),
          pltpu.VMEM((16,), jnp.float32),
          pltpu.VMEM((ROWS_A, 16), jnp.float32),
          pltpu.VMEM_SHARED((N, 16), jnp.float32),
          pltpu.SemaphoreType.DMA,
          pltpu.SemaphoreType.DMA,
          pltpu.SemaphoreType.DMA,
          pltpu.SemaphoreType.DMA,
          pltpu.SemaphoreType.DMA,
          pltpu.SemaphoreType.DMA,
      ])
  def k(aas_hbm, aad_hbm, m_hbm, src_hbm, dst_hbm, ex_hbm, den_hbm,
        sbig, dbig, ab0, ab1, db0, db1, eb0, eb1, mb, zb, den_sh,
        gsem0, gsem1, wsem0, wsem1, ssem0, ssem1):
    cid = lax.axis_index("c")
    sid = lax.axis_index("s")
    wid = cid * NS + sid
    ab = (ab0, ab1)
    db = (db0, db1)
    eb = (eb0, eb1)
    gsem = (gsem0, gsem1)
    wsem = (wsem0, wsem1)
    ssem = (ssem0, ssem1)

    @pl.loop(0, ROWS_A)
    def _(z):
      zb[z, :] = jnp.zeros((16,), jnp.float32)

    row0 = sid * ROWS_A

    @pl.when(sid < NS - 1)
    def _():
      pltpu.sync_copy(zb, den_sh.at[pl.ds(row0, ROWS_A)])

    @pl.when(sid == NS - 1)
    def _():
      pltpu.sync_copy(zb.at[pl.ds(0, ROWS_LAST)],
                      den_sh.at[pl.ds(row0, ROWS_LAST)])

    pltpu.sync_copy(m_hbm, mb)
    plsc.subcore_barrier()

    mraw = mb[...]
    mv = jnp.where(mraw > 0, mraw, 0.2 * mraw)
    base_p = wid * EPW

    pltpu.sync_copy(src_hbm.at[pl.ds(wid * NCH1, NCH1)], sbig)
    pltpu.sync_copy(dst_hbm.at[pl.ds(wid * NCH1, NCH1)], dbig)
    pltpu.async_copy(aas_hbm.at[sbig.at[0]], ab0, gsem0)
    pltpu.async_copy(aad_hbm.at[dbig.at[0]], db0, gsem0)

    @pl.loop(0, NCH1 // 2)
    def _(ii):
      for b in (0, 1):
        i = ii * 2 + b
        nb = 1 - b
        pltpu.make_async_copy(aas_hbm.at[sbig.at[0]], ab[b], gsem[b]).wait()
        pltpu.make_async_copy(aad_hbm.at[dbig.at[0]], db[b], gsem[b]).wait()

        if b == 0:
          pltpu.async_copy(aas_hbm.at[sbig.at[i + 1]], ab[nb], gsem[nb])
          pltpu.async_copy(aad_hbm.at[dbig.at[i + 1]], db[nb], gsem[nb])
        else:
          @pl.when(ii < NCH1 // 2 - 1)
          def _():
            pltpu.async_copy(aas_hbm.at[sbig.at[i + 1]], ab[nb], gsem[nb])
            pltpu.async_copy(aad_hbm.at[dbig.at[i + 1]], db[nb], gsem[nb])

        @pl.when(ii >= 1)
        def _():
          pltpu.make_async_copy(eb[b], ex_hbm.at[pl.ds(base_p, CH1)],
                                wsem[b]).wait()
          pltpu.make_async_copy(eb[b], den_sh.at[dbig.at[0]],
                                ssem[b]).wait()

        @pl.loop(0, CH1)
        def _(c):
          v = ab[b][c, :] + db[b][c, :]
          v = jnp.where(v > 0, v, 0.2 * v)
          eb[b][c, :] = jnp.exp(v - mv)

        pltpu.async_copy(eb[b], ex_hbm.at[pl.ds(base_p + i * CH1, CH1)],
                         wsem[b])
        pltpu.async_copy(eb[b], den_sh.at[dbig.at[i]], ssem[b], add=True)

    for b in (0, 1):
      pltpu.make_async_copy(eb[b], ex_hbm.at[pl.ds(base_p, CH1)],
                            wsem[b]).wait()
      pltpu.make_async_copy(eb[b], den_sh.at[dbig.at[0]], ssem[b]).wait()

    pltpu.sync_copy(zb.at[pl.ds(0, PAD_W)],
                    ex_hbm.at[pl.ds(base_p + E_PER_W, PAD_W)])

    plsc.subcore_barrier()

    @pl.when(sid < NS - 1)
    def _():
      pltpu.sync_copy(den_sh.at[pl.ds(row0, ROWS_A)],
                      den_hbm.at[pl.ds(cid * N + row0, ROWS_A)])

    @pl.when(sid == NS - 1)
    def _():
      pltpu.sync_copy(den_sh.at[pl.ds(row0, ROWS_LAST)],
                      den_hbm.at[pl.ds(cid * N + row0, ROWS_LAST)])

  return k(aas, aad, m_raw, src40, dst40)


def _den_recip(den):
  d0 = den[:N]
  d1 = den[N:]

  def body(a_ref, b_ref, o_ref):
    o_ref[...] = 1.0 / (a_ref[...] + b_ref[...] + 1e-16)

  return pl.pallas_call(
      body, grid=(GRID_N,),
      in_specs=[pl.BlockSpec((BN_BLK, 16), lambda i: (i, 0)),
                pl.BlockSpec((BN_BLK, 16), lambda i: (i, 0))],
      out_specs=pl.BlockSpec((BN_BLK, 16), lambda i: (i, 0)),
      out_shape=jax.ShapeDtypeStruct((N, 16), jnp.float32))(d0, d1)


def _sc_pass2(h, exP, rden, srcP, dstP):
  @functools.partial(
      pl.kernel,
      out_type=jax.ShapeDtypeStruct((2 * N, HID), jnp.float32),
      mesh=_sc_mesh(),
      compiler_params=_SC_PARAMS,
      scratch_types=[
          pltpu.VMEM((SUPC, 16), jnp.int32),
          pltpu.VMEM((SUPC, 16), jnp.int32),
          pltpu.VMEM((SUPC, 16), jnp.int32),
          pltpu.VMEM((SUPC, 16), jnp.int32),
          pltpu.VMEM((CHK, 16), jnp.float32),
          pltpu.VMEM((CHK, 16), jnp.float32),
          pltpu.VMEM((CHK, 16), jnp.float32),
          pltpu.VMEM((CHK, 16), jnp.float32),
          pltpu.VMEM((CHK, HEADS * HID), jnp.float32),
          pltpu.VMEM((CHK, HEADS * HID), jnp.float32),
          pltpu.VMEM((CHK, HID), jnp.float32),
          pltpu.VMEM((CHK, HID), jnp.float32),
          pltpu.VMEM_SHARED((N, HID), jnp.float32),
          pltpu.SemaphoreType.DMA,
          pltpu.SemaphoreType.DMA,
          pltpu.SemaphoreType.DMA,
          pltpu.SemaphoreType.DMA,
          pltpu.SemaphoreType.DMA,
          pltpu.SemaphoreType.DMA,
      ])
  def k(h_hbm, ex_hbm, rd_hbm, src_hbm, dst_hbm, agg_hbm,
        sbuf0, sbuf1, dbuf0, dbuf1, eb0, eb1, rb0, rb1, hb0, hb1,
        mb0, mb1, agg_sh, isem0, isem1, gsem0, gsem1, ssem0, ssem1):
    cid = lax.axis_index("c")
    sid = lax.axis_index("s")
    wid = cid * NS + sid
    sbuf = (sbuf0, sbuf1)
    dbuf = (dbuf0, dbuf1)
    eb = (eb0, eb1)
    rb = (rb0, rb1)
    hb = (hb0, hb1)
    mb = (mb0, mb1)
    isem = (isem0, isem1)
    gsem = (gsem0, gsem1)
    ssem = (ssem0, ssem1)

    @pl.loop(0, CHK)
    def _(z):
      for kk2 in range(HID // 16):
        mb0[z, pl.ds(kk2 * 16, 16)] = jnp.zeros((16,), jnp.float32)

    row0 = sid * ROWS_A

    @pl.when(sid < NS - 1)
    def _():
      @pl.loop(0, 39)
      def _(kk2):
        pltpu.sync_copy(mb0, agg_sh.at[pl.ds(row0 + kk2 * CHK, CHK)])
      pltpu.sync_copy(mb0.at[pl.ds(0, 8)], agg_sh.at[pl.ds(row0 + 624, 8)])

    @pl.when(sid == NS - 1)
    def _():
      @pl.loop(0, 32)
      def _(kk2):
        pltpu.sync_copy(mb0, agg_sh.at[pl.ds(row0 + kk2 * CHK, CHK)])
      pltpu.sync_copy(mb0.at[pl.ds(0, 8)], agg_sh.at[pl.ds(row0 + 512, 8)])

    plsc.subcore_barrier()

    ebase = wid * EPW
    irow0 = wid * (EPW // 16)

    def issue_chunk(off, sb_ref, db_ref, jrow, slot):
      pltpu.async_copy(ex_hbm.at[pl.ds(off, CHK)], eb[slot], gsem[slot])
      pltpu.async_copy(rd_hbm.at[db_ref.at[jrow]], rb[slot], gsem[slot])
      pltpu.async_copy(h_hbm.at[sb_ref.at[jrow]], hb[slot], gsem[slot])

    def wait_chunk(sb_ref, db_ref, slot):
      pltpu.make_async_copy(ex_hbm.at[pl.ds(ebase, CHK)], eb[slot],
                            gsem[slot]).wait()
      pltpu.make_async_copy(rd_hbm.at[db_ref.at[0]], rb[slot],
                            gsem[slot]).wait()
      pltpu.make_async_copy(h_hbm.at[sb_ref.at[0]], hb[slot],
                            gsem[slot]).wait()

    pltpu.sync_copy(src_hbm.at[pl.ds(irow0, SUPC)], sbuf0)
    pltpu.sync_copy(dst_hbm.at[pl.ds(irow0, SUPC)], dbuf0)
    issue_chunk(ebase, sbuf0, dbuf0, 0, 0)

    @pl.loop(0, NSUP // 2)
    def _(tp):
      for ss in (0, 1):
        t = tp * 2 + ss
        tbase = ebase + t * (SUPC * CHK)
        sb = sbuf[ss]
        db = dbuf[ss]

        @pl.loop(0, SUPC // 2)
        def _(jj):
          for b in (0, 1):
            j = jj * 2 + b
            nb = 1 - b
            wait_chunk(sb, db, b)

            if b == 0:
              issue_chunk(tbase + (j + 1) * CHK, sb, db, j + 1, nb)
            else:
              @pl.when(jj < SUPC // 2 - 1)
              def _():
                issue_chunk(tbase + (j + 1) * CHK, sb, db, j + 1, nb)

              @pl.when((jj == SUPC // 2 - 1) & (t < NSUP - 1))
              def _():
                pltpu.make_async_copy(
                    src_hbm.at[pl.ds(irow0, SUPC)], sbuf[1 - ss],
                    isem[1 - ss]).wait()
                pltpu.make_async_copy(
                    dst_hbm.at[pl.ds(irow0, SUPC)], dbuf[1 - ss],
                    isem[1 - ss]).wait()
                issue_chunk(tbase + SUPC * CHK, sbuf[1 - ss], dbuf[1 - ss],
                            0, nb)

            @pl.when((t > 0) | (jj >= 1))
            def _():
              pltpu.make_async_copy(mb[b], agg_sh.at[db.at[j]],
                                    ssem[b]).wait()

            @pl.loop(0, CHK)
            def _(c):
              wv = eb[b][c, :] * rb[b][c, :]
              ws = [jnp.full((16,), wv[hh], jnp.float32)
                    for hh in range(HEADS)]
              for kk2 in range(HID // 16):
                acc = ws[0] * hb[b][c, pl.ds(kk2 * 16, 16)]
                for hh in range(1, HEADS):
                  acc = acc + ws[hh] * hb[b][c, pl.ds(hh * HID + kk2 * 16, 16)]
                mb[b][c, pl.ds(kk2 * 16, 16)] = acc

            pltpu.async_copy(mb[b], agg_sh.at[db.at[j]], ssem[b], add=True)

            if b == 1:
              @pl.when((jj == 0) & (t < NSUP - 1))
              def _():
                nrow = irow0 + (t + 1) * SUPC
                pltpu.async_copy(src_hbm.at[pl.ds(nrow, SUPC)],
                                 sbuf[1 - ss], isem[1 - ss])
                pltpu.async_copy(dst_hbm.at[pl.ds(nrow, SUPC)],
                                 dbuf[1 - ss], isem[1 - ss])

    pltpu.make_async_copy(mb0, agg_sh.at[dbuf1.at[0]], ssem0).wait()
    pltpu.make_async_copy(mb1, agg_sh.at[dbuf1.at[0]], ssem1).wait()

    plsc.subcore_barrier()

    @pl.when(sid < NS - 1)
    def _():
      pltpu.sync_copy(agg_sh.at[pl.ds(row0, ROWS_A)],
                      agg_hbm.at[pl.ds(cid * N + row0, ROWS_A)])

    @pl.when(sid == NS - 1)
    def _():
      pltpu.sync_copy(agg_sh.at[pl.ds(row0, ROWS_LAST)],
                      agg_hbm.at[pl.ds(cid * N + row0, ROWS_LAST)])

  return k(h, exP, rden, srcP, dstP)


def _block_diag(att):
  flat = att.reshape(-1)
  r = jnp.arange(HEADS * HID)
  B = jnp.zeros((HEADS * HID, 16), jnp.float32)
  return B.at[r, r // HID].set(flat)


def _gat_layer(xin, W, att_src, att_dst, src40, dst40, srcP, dstP,
               skip_W=None, skip_b=None):
  Bs = _block_diag(att_src)
  Bd = _block_diag(att_dst)
  if skip_W is not None:
    h, aas, aad, ms, md, skip = _dense_block(xin, W, Bs, Bd, skip_W, skip_b)
  else:
    h, aas, aad, ms, md = _dense_block(xin, W, Bs, Bd)
    skip = None
  m_raw = (ms + md).reshape(16)
  exP, den = _sc_pass1(aas, aad, m_raw, src40, dst40)
  rden = _den_recip(den)
  agg = _sc_pass2(h, exP, rden, srcP, dstP)
  return agg[:N], agg[N:], skip


def kernel(x, edge_index, batch_idx, W1, att_src1, att_dst1, bias1, bn1_g,
           bn1_b, skip_W, skip_b, W2, att_src2, att_dst2, bias2, bn2_g,
           bn2_b):
  src = edge_index[0].astype(jnp.int32)
  dst = edge_index[1].astype(jnp.int32)
  src40 = src.reshape(IR1, CH1)
  dst40 = dst.reshape(IR1, CH1)
  srcP = jnp.pad(src.reshape(NW, E_PER_W),
                 ((0, 0), (0, PAD_W))).reshape(EP // 16, 16)
  dstP = jnp.pad(dst.reshape(NW, E_PER_W),
                 ((0, 0), (0, PAD_W))).reshape(EP // 16, 16)

  p0, p1, skip = _gat_layer(x, W1, att_src1, att_dst1, src40, dst40,
                            srcP, dstP, skip_W, skip_b)
  hmid = _post_bn_gelu(p0, p1, skip, bias1, bn1_g, bn1_b)

  q0, q1, _ = _gat_layer(hmid, W2, att_src2, att_dst2, src40, dst40,
                         srcP, dstP)
  batch_row = batch_idx.astype(jnp.int32).reshape(GRID_N, 1, BN_BLK)
  return _post_bn_gelu_pool(q0, q1, batch_row, bias2, bn2_g, bn2_b)


# speedup vs baseline: 33.2202x; 1.3885x over previous
import functools

import jax
import jax.numpy as jnp
import numpy as np
from jax import lax
from jax.experimental import pallas as pl
from jax.experimental.pallas import tpu as pltpu
from jax.experimental.pallas import tpu_sc as plsc

N = 10000
E = 320000
D = 128
HID = 128
HEADS = 8
NG = 64

NC = 2
NS = 16
NW = NC * NS
E_PER_W = E // NW
CH1 = 40
NCH1 = E_PER_W // CH1
IR1 = E // CH1

CHK = 32
SUPC = 40
NSUP = 8
EPW = CHK * SUPC * NSUP
EP = EPW * NW
PAD_W = EPW - E_PER_W
ROWS_A = 632
ROWS_LAST = N - 15 * ROWS_A

BN_BLK = 2000
GRID_N = N // BN_BLK


def _dense_block(x, W, Bs, Bd, skip_W=None, skip_b=None):
  D_in = x.shape[1]
  with_skip = skip_W is not None

  def body(*refs):
    if with_skip:
      (x_ref, w_ref, bs_ref, bd_ref, sw_ref, sb_ref,
       h_ref, as_ref, ad_ref, ms_ref, md_ref, sk_ref) = refs
    else:
      (x_ref, w_ref, bs_ref, bd_ref,
       h_ref, as_ref, ad_ref, ms_ref, md_ref) = refs
    i = pl.program_id(0)
    xb = x_ref[...]
    hb = jnp.dot(xb, w_ref[...], preferred_element_type=jnp.float32)
    h_ref[...] = hb.astype(jnp.bfloat16)
    a_s = jnp.dot(hb, bs_ref[...], preferred_element_type=jnp.float32)
    a_d = jnp.dot(hb, bd_ref[...], preferred_element_type=jnp.float32)
    as_ref[...] = a_s
    ad_ref[...] = a_d

    @pl.when(i == 0)
    def _():
      ms_ref[...] = jnp.full((1, 16), -1e30, jnp.float32)
      md_ref[...] = jnp.full((1, 16), -1e30, jnp.float32)

    ms_ref[...] = jnp.maximum(ms_ref[...], jnp.max(a_s, axis=0, keepdims=True))
    md_ref[...] = jnp.maximum(md_ref[...], jnp.max(a_d, axis=0, keepdims=True))
    if with_skip:
      sk_ref[...] = (jnp.dot(xb, sw_ref[...], preferred_element_type=jnp.float32)
                     + sb_ref[...])

  out_shape = [
      jax.ShapeDtypeStruct((N, HEADS * HID), jnp.bfloat16),
      jax.ShapeDtypeStruct((N, 16), jnp.float32),
      jax.ShapeDtypeStruct((N, 16), jnp.float32),
      jax.ShapeDtypeStruct((1, 16), jnp.float32),
      jax.ShapeDtypeStruct((1, 16), jnp.float32),
  ]
  out_specs = [
      pl.BlockSpec((BN_BLK, HEADS * HID), lambda i: (i, 0)),
      pl.BlockSpec((BN_BLK, 16), lambda i: (i, 0)),
      pl.BlockSpec((BN_BLK, 16), lambda i: (i, 0)),
      pl.BlockSpec((1, 16), lambda i: (0, 0)),
      pl.BlockSpec((1, 16), lambda i: (0, 0)),
  ]
  in_specs = [
      pl.BlockSpec((BN_BLK, D_in), lambda i: (i, 0)),
      pl.BlockSpec((D_in, HEADS * HID), lambda i: (0, 0)),
      pl.BlockSpec((HEADS * HID, 16), lambda i: (0, 0)),
      pl.BlockSpec((HEADS * HID, 16), lambda i: (0, 0)),
  ]
  args = [x, W, Bs, Bd]
  if with_skip:
    in_specs += [pl.BlockSpec((D_in, HID), lambda i: (0, 0)),
                 pl.BlockSpec((1, HID), lambda i: (0, 0))]
    args += [skip_W, skip_b.reshape(1, HID)]
    out_shape.append(jax.ShapeDtypeStruct((N, HID), jnp.float32))
    out_specs.append(pl.BlockSpec((BN_BLK, HID), lambda i: (i, 0)))

  return pl.pallas_call(
      body, grid=(GRID_N,), in_specs=in_specs, out_specs=out_specs,
      out_shape=out_shape)(*args)


def _erf(x):
  a1, a2, a3, a4, a5 = (0.254829592, -0.284496736, 1.421413741,
                        -1.453152027, 1.061405429)
  p = 0.3275911
  s = jnp.sign(x)
  ax = jnp.abs(x)
  t = 1.0 / (1.0 + p * ax)
  poly = ((((a5 * t + a4) * t + a3) * t + a2) * t + a1) * t
  y = 1.0 - poly * jnp.exp(-ax * ax)
  return s * y


def _gelu(x):
  return x * 0.5 * (1.0 + _erf(x * np.float32(1.0 / np.sqrt(2.0))))


def _post_bn_gelu(p0, p1, skip, bias, g, b):
  def body(p0_ref, p1_ref, sk_ref, bias_ref, g_ref, b_ref, out_ref,
           sums, sumsq):
    ph = pl.program_id(0)
    i = pl.program_id(1)
    gv = (p0_ref[...] + p1_ref[...]) * np.float32(1.0 / HEADS) + bias_ref[...]

    @pl.when(ph == 0)
    def _():
      @pl.when(i == 0)
      def _():
        sums[...] = jnp.zeros((1, HID), jnp.float32)
        sumsq[...] = jnp.zeros((1, HID), jnp.float32)
      sums[...] += jnp.sum(gv, axis=0, keepdims=True)
      sumsq[...] += jnp.sum(gv * gv, axis=0, keepdims=True)

    @pl.when(ph == 1)
    def _():
      mu = sums[...] * np.float32(1.0 / N)
      var = sumsq[...] * np.float32(1.0 / N) - mu * mu
      xn = (gv - mu) * lax.rsqrt(var + 1e-5) * g_ref[...] + b_ref[...]
      out_ref[...] = _gelu(xn + sk_ref[...])

  return pl.pallas_call(
      body, grid=(2, GRID_N),
      in_specs=[
          pl.BlockSpec((BN_BLK, HID), lambda p, i: (i, 0)),
          pl.BlockSpec((BN_BLK, HID), lambda p, i: (i, 0)),
          pl.BlockSpec((BN_BLK, HID), lambda p, i: (i, 0)),
          pl.BlockSpec((1, HID), lambda p, i: (0, 0)),
          pl.BlockSpec((1, HID), lambda p, i: (0, 0)),
          pl.BlockSpec((1, HID), lambda p, i: (0, 0)),
      ],
      out_specs=pl.BlockSpec((BN_BLK, HID), lambda p, i: (i, 0)),
      out_shape=jax.ShapeDtypeStruct((N, HID), jnp.float32),
      scratch_shapes=[pltpu.VMEM((1, HID), jnp.float32),
                      pltpu.VMEM((1, HID), jnp.float32)],
  )(p0, p1, skip, bias.reshape(1, HID), g.reshape(1, HID), b.reshape(1, HID))


def _post_bn_gelu_pool(p0, p1, batch_row, bias, g, b):
  def body(p0_ref, p1_ref, bi_ref, bias_ref, g_ref, b_ref, out_ref,
           sums, sumsq, pool, cnt):
    ph = pl.program_id(0)
    i = pl.program_id(1)
    gv = (p0_ref[...] + p1_ref[...]) * np.float32(1.0 / HEADS) + bias_ref[...]

    @pl.when(ph == 0)
    def _():
      @pl.when(i == 0)
      def _():
        sums[...] = jnp.zeros((1, HID), jnp.float32)
        sumsq[...] = jnp.zeros((1, HID), jnp.float32)
      sums[...] += jnp.sum(gv, axis=0, keepdims=True)
      sumsq[...] += jnp.sum(gv * gv, axis=0, keepdims=True)

    @pl.when(ph == 1)
    def _():
      mu = sums[...] * np.float32(1.0 / N)
      var = sumsq[...] * np.float32(1.0 / N) - mu * mu
      xn = (gv - mu) * lax.rsqrt(var + 1e-5) * g_ref[...] + b_ref[...] + gv
      o = _gelu(xn)
      ids = lax.broadcasted_iota(jnp.int32, (NG, BN_BLK), 0)
      oh = (ids == bi_ref[...].reshape(1, BN_BLK)).astype(jnp.float32)

      @pl.when(i == 0)
      def _():
        pool[...] = jnp.zeros((NG, HID), jnp.float32)
        cnt[...] = jnp.zeros((NG, HID), jnp.float32)

      pool[...] += lax.dot_general(oh, o, (((1,), (0,)), ((), ())),
                                   preferred_element_type=jnp.float32)
      cnt[...] += jnp.broadcast_to(jnp.sum(oh, axis=1, keepdims=True),
                                   (NG, HID))

      @pl.when(i == GRID_N - 1)
      def _():
        out_ref[...] = pool[...] / jnp.maximum(cnt[...], 1.0)

  return pl.pallas_call(
      body, grid=(2, GRID_N),
      in_specs=[
          pl.BlockSpec((BN_BLK, HID), lambda p, i: (i, 0)),
          pl.BlockSpec((BN_BLK, HID), lambda p, i: (i, 0)),
          pl.BlockSpec((1, 1, BN_BLK), lambda p, i: (i, 0, 0)),
          pl.BlockSpec((1, HID), lambda p, i: (0, 0)),
          pl.BlockSpec((1, HID), lambda p, i: (0, 0)),
          pl.BlockSpec((1, HID), lambda p, i: (0, 0)),
      ],
      out_specs=pl.BlockSpec((NG, HID), lambda p, i: (0, 0)),
      out_shape=jax.ShapeDtypeStruct((NG, HID), jnp.float32),
      scratch_shapes=[pltpu.VMEM((1, HID), jnp.float32),
                      pltpu.VMEM((1, HID), jnp.float32),
                      pltpu.VMEM((NG, HID), jnp.float32),
                      pltpu.VMEM((NG, HID), jnp.float32)],
  )(p0, p1, batch_row, bias.reshape(1, HID), g.reshape(1, HID),
    b.reshape(1, HID))


def _sc_mesh():
  return plsc.VectorSubcoreMesh(core_axis_name="c", subcore_axis_name="s")


_SC_PARAMS = pltpu.CompilerParams(use_tc_tiling_on_sc=False,
                                  needs_layout_passes=False)


def _sc_pass1(aas, aad, m_raw, src40, dst40):
  @functools.partial(
      pl.kernel,
      out_type=(jax.ShapeDtypeStruct((EP, 16), jnp.float32),
                jax.ShapeDtypeStruct((2 * N, 16), jnp.float32)),
      mesh=_sc_mesh(),
      compiler_params=_SC_PARAMS,
      scratch_types=[
          pltpu.VMEM((NCH1, CH1), jnp.int32),
          pltpu.VMEM((NCH1, CH1), jnp.int32),
          pltpu.VMEM((CH1, 16), jnp.float32),
          pltpu.VMEM((CH1, 16), jnp.float32),
          pltpu.VMEM((CH1, 16), jnp.float32),
          pltpu.VMEM((CH1, 16), jnp.float32),
          pltpu.VMEM((CH1, 16), jnp.float32),
          pltpu.VMEM((CH1, 16), jnp.float32),
          pltpu.VMEM((16,), jnp.float32),
          pltpu.VMEM((ROWS_A, 16), jnp.float32),
          pltpu.VMEM_SHARED((N, 16), jnp.float32),
          pltpu.SemaphoreType.DMA,
          pltpu.SemaphoreType.DMA,
          pltpu.SemaphoreType.DMA,
          pltpu.SemaphoreType.DMA,
          pltpu.SemaphoreType.DMA,
          pltpu.SemaphoreType.DMA,
      ])
  def k(aas_hbm, aad_hbm, m_hbm, src_hbm, dst_hbm, ex_hbm, den_hbm,
        sbig, dbig, ab0, ab1, db0, db1, eb0, eb1, mb, zb, den_sh,
        gsem0, gsem1, wsem0, wsem1, ssem0, ssem1):
    cid = lax.axis_index("c")
    sid = lax.axis_index("s")
    wid = cid * NS + sid
    ab = (ab0, ab1)
    db = (db0, db1)
    eb = (eb0, eb1)
    gsem = (gsem0, gsem1)
    wsem = (wsem0, wsem1)
    ssem = (ssem0, ssem1)

    @pl.loop(0, ROWS_A)
    def _(z):
      zb[z, :] = jnp.zeros((16,), jnp.float32)

    row0 = sid * ROWS_A

    @pl.when(sid < NS - 1)
    def _():
      pltpu.sync_copy(zb, den_sh.at[pl.ds(row0, ROWS_A)])

    @pl.when(sid == NS - 1)
    def _():
      pltpu.sync_copy(zb.at[pl.ds(0, ROWS_LAST)],
                      den_sh.at[pl.ds(row0, ROWS_LAST)])

    pltpu.sync_copy(m_hbm, mb)
    plsc.subcore_barrier()

    mraw = mb[...]
    mv = jnp.where(mraw > 0, mraw, 0.2 * mraw)
    base_p = wid * EPW

    pltpu.sync_copy(src_hbm.at[pl.ds(wid * NCH1, NCH1)], sbig)
    pltpu.sync_copy(dst_hbm.at[pl.ds(wid * NCH1, NCH1)], dbig)
    pltpu.async_copy(aas_hbm.at[sbig.at[0]], ab0, gsem0)
    pltpu.async_copy(aad_hbm.at[dbig.at[0]], db0, gsem0)

    @pl.loop(0, NCH1 // 2)
    def _(ii):
      for b in (0, 1):
        i = ii * 2 + b
        nb = 1 - b
        pltpu.make_async_copy(aas_hbm.at[sbig.at[0]], ab[b], gsem[b]).wait()
        pltpu.make_async_copy(aad_hbm.at[dbig.at[0]], db[b], gsem[b]).wait()

        if b == 0:
          pltpu.async_copy(aas_hbm.at[sbig.at[i + 1]], ab[nb], gsem[nb])
          pltpu.async_copy(aad_hbm.at[dbig.at[i + 1]], db[nb], gsem[nb])
        else:
          @pl.when(ii < NCH1 // 2 - 1)
          def _():
            pltpu.async_copy(aas_hbm.at[sbig.at[i + 1]], ab[nb], gsem[nb])
            pltpu.async_copy(aad_hbm.at[dbig.at[i + 1]], db[nb], gsem[nb])

        @pl.when(ii >= 1)
        def _():
          pltpu.make_async_copy(eb[b], ex_hbm.at[pl.ds(base_p, CH1)],
                                wsem[b]).wait()
          pltpu.make_async_copy(eb[b], den_sh.at[dbig.at[0]],
                                ssem[b]).wait()

        @pl.loop(0, CH1)
        def _(c):
          v = ab[b][c, :] + db[b][c, :]
          v = jnp.where(v > 0, v, 0.2 * v)
          eb[b][c, :] = jnp.exp(v - mv)

        pltpu.async_copy(eb[b], ex_hbm.at[pl.ds(base_p + i * CH1, CH1)],
                         wsem[b])
        pltpu.async_copy(eb[b], den_sh.at[dbig.at[i]], ssem[b], add=True)

    for b in (0, 1):
      pltpu.make_async_copy(eb[b], ex_hbm.at[pl.ds(base_p, CH1)],
                            wsem[b]).wait()
      pltpu.make_async_copy(eb[b], den_sh.at[dbig.at[0]], ssem[b]).wait()

    pltpu.sync_copy(zb.at[pl.ds(0, PAD_W)],
                    ex_hbm.at[pl.ds(base_p + E_PER_W, PAD_W)])

    plsc.subcore_barrier()

    @pl.when(sid < NS - 1)
    def _():
      pltpu.sync_copy(den_sh.at[pl.ds(row0, ROWS_A)],
                      den_hbm.at[pl.ds(cid * N + row0, ROWS_A)])

    @pl.when(sid == NS - 1)
    def _():
      pltpu.sync_copy(den_sh.at[pl.ds(row0, ROWS_LAST)],
                      den_hbm.at[pl.ds(cid * N + row0, ROWS_LAST)])

  return k(aas, aad, m_raw, src40, dst40)


def _den_recip(den):
  d0 = den[:N]
  d1 = den[N:]

  def body(a_ref, b_ref, o_ref):
    o_ref[...] = 1.0 / (a_ref[...] + b_ref[...] + 1e-16)

  return pl.pallas_call(
      body, grid=(GRID_N,),
      in_specs=[pl.BlockSpec((BN_BLK, 16), lambda i: (i, 0)),
                pl.BlockSpec((BN_BLK, 16), lambda i: (i, 0))],
      out_specs=pl.BlockSpec((BN_BLK, 16), lambda i: (i, 0)),
      out_shape=jax.ShapeDtypeStruct((N, 16), jnp.float32))(d0, d1)


def _sc_pass2(h, exP, rden, srcP, dstP):
  @functools.partial(
      pl.kernel,
      out_type=jax.ShapeDtypeStruct((2 * N, HID), jnp.float32),
      mesh=_sc_mesh(),
      compiler_params=_SC_PARAMS,
      scratch_types=[
          pltpu.VMEM((SUPC, CHK), jnp.int32),
          pltpu.VMEM((SUPC, CHK), jnp.int32),
          pltpu.VMEM((SUPC, CHK), jnp.int32),
          pltpu.VMEM((SUPC, CHK), jnp.int32),
          pltpu.VMEM((CHK, 16), jnp.float32),
          pltpu.VMEM((CHK, 16), jnp.float32),
          pltpu.VMEM((CHK, 16), jnp.float32),
          pltpu.VMEM((CHK, 16), jnp.float32),
          pltpu.VMEM((CHK, HEADS * HID), jnp.bfloat16),
          pltpu.VMEM((CHK, HEADS * HID), jnp.bfloat16),
          pltpu.VMEM((CHK, HID), jnp.float32),
          pltpu.VMEM((CHK, HID), jnp.float32),
          pltpu.VMEM_SHARED((N, HID), jnp.float32),
          pltpu.SemaphoreType.DMA,
          pltpu.SemaphoreType.DMA,
          pltpu.SemaphoreType.DMA,
          pltpu.SemaphoreType.DMA,
          pltpu.SemaphoreType.DMA,
          pltpu.SemaphoreType.DMA,
      ])
  def k(h_hbm, ex_hbm, rd_hbm, src_hbm, dst_hbm, agg_hbm,
        sbuf0, sbuf1, dbuf0, dbuf1, eb0, eb1, rb0, rb1, hb0, hb1,
        mb0, mb1, agg_sh, isem0, isem1, gsem0, gsem1, ssem0, ssem1):
    cid = lax.axis_index("c")
    sid = lax.axis_index("s")
    wid = cid * NS + sid
    sbuf = (sbuf0, sbuf1)
    dbuf = (dbuf0, dbuf1)
    eb = (eb0, eb1)
    rb = (rb0, rb1)
    hb = (hb0, hb1)
    mb = (mb0, mb1)
    isem = (isem0, isem1)
    gsem = (gsem0, gsem1)
    ssem = (ssem0, ssem1)

    @pl.loop(0, CHK)
    def _(z):
      for kk2 in range(HID // 16):
        mb0[z, pl.ds(kk2 * 16, 16)] = jnp.zeros((16,), jnp.float32)

    row0 = sid * ROWS_A

    @pl.when(sid < NS - 1)
    def _():
      @pl.loop(0, 19)
      def _(kk2):
        pltpu.sync_copy(mb0, agg_sh.at[pl.ds(row0 + kk2 * CHK, CHK)])
      pltpu.sync_copy(mb0.at[pl.ds(0, 24)], agg_sh.at[pl.ds(row0 + 608, 24)])

    @pl.when(sid == NS - 1)
    def _():
      @pl.loop(0, 16)
      def _(kk2):
        pltpu.sync_copy(mb0, agg_sh.at[pl.ds(row0 + kk2 * CHK, CHK)])
      pltpu.sync_copy(mb0.at[pl.ds(0, 8)], agg_sh.at[pl.ds(row0 + 512, 8)])

    plsc.subcore_barrier()

    ebase = wid * EPW
    irow0 = wid * (EPW // CHK)

    def issue_chunk(off, sb_ref, db_ref, jrow, slot):
      pltpu.async_copy(ex_hbm.at[pl.ds(off, CHK)], eb[slot], gsem[slot])
      pltpu.async_copy(rd_hbm.at[db_ref.at[jrow]], rb[slot], gsem[slot])
      pltpu.async_copy(h_hbm.at[sb_ref.at[jrow]], hb[slot], gsem[slot])

    def wait_chunk(sb_ref, db_ref, slot):
      pltpu.make_async_copy(ex_hbm.at[pl.ds(ebase, CHK)], eb[slot],
                            gsem[slot]).wait()
      pltpu.make_async_copy(rd_hbm.at[db_ref.at[0]], rb[slot],
                            gsem[slot]).wait()
      pltpu.make_async_copy(h_hbm.at[sb_ref.at[0]], hb[slot],
                            gsem[slot]).wait()

    pltpu.sync_copy(src_hbm.at[pl.ds(irow0, SUPC)], sbuf0)
    pltpu.sync_copy(dst_hbm.at[pl.ds(irow0, SUPC)], dbuf0)
    issue_chunk(ebase, sbuf0, dbuf0, 0, 0)

    @pl.loop(0, NSUP // 2)
    def _(tp):
      for ss in (0, 1):
        t = tp * 2 + ss
        tbase = ebase + t * (SUPC * CHK)
        sb = sbuf[ss]
        db = dbuf[ss]

        @pl.loop(0, SUPC // 2)
        def _(jj):
          for b in (0, 1):
            j = jj * 2 + b
            nb = 1 - b

            if b == 0:
              issue_chunk(tbase + (j + 1) * CHK, sb, db, j + 1, nb)
            else:
              @pl.when(jj < SUPC // 2 - 1)
              def _():
                issue_chunk(tbase + (j + 1) * CHK, sb, db, j + 1, nb)

              @pl.when((jj == SUPC // 2 - 1) & (t < NSUP - 1))
              def _():
                pltpu.make_async_copy(
                    src_hbm.at[pl.ds(irow0, SUPC)], sbuf[1 - ss],
                    isem[1 - ss]).wait()
                pltpu.make_async_copy(
                    dst_hbm.at[pl.ds(irow0, SUPC)], dbuf[1 - ss],
                    isem[1 - ss]).wait()
                issue_chunk(tbase + SUPC * CHK, sbuf[1 - ss], dbuf[1 - ss],
                            0, nb)

            wait_chunk(sb, db, b)

            @pl.when((t > 0) | (jj >= 1))
            def _():
              pltpu.make_async_copy(mb[b], agg_sh.at[db.at[j]],
                                    ssem[b]).wait()

            @pl.loop(0, CHK)
            def _(c):
              wv = eb[b][c, :] * rb[b][c, :]
              ws = [jnp.full((16,), wv[hh], jnp.float32)
                    for hh in range(HEADS)]
              for m in range(HID // 32):
                v32 = hb[b][c, pl.ds((4 * 0 + m) * 32, 32)]
                pa, pb = plsc.unpack(v32, format=plsc.PackFormat.INTERLEAVED)
                acc0 = ws[0] * pa
                acc1 = ws[0] * pb
                for hh in range(1, HEADS):
                  v32 = hb[b][c, pl.ds((4 * hh + m) * 32, 32)]
                  pa, pb = plsc.unpack(v32,
                                       format=plsc.PackFormat.INTERLEAVED)
                  acc0 = acc0 + ws[hh] * pa
                  acc1 = acc1 + ws[hh] * pb
                mb[b][c, pl.ds(2 * m * 16, 16)] = acc0
                mb[b][c, pl.ds((2 * m + 1) * 16, 16)] = acc1

            pltpu.async_copy(mb[b], agg_sh.at[db.at[j]], ssem[b], add=True)

            if b == 1:
              @pl.when((jj == 0) & (t < NSUP - 1))
              def _():
                nrow = irow0 + (t + 1) * SUPC
                pltpu.async_copy(src_hbm.at[pl.ds(nrow, SUPC)],
                                 sbuf[1 - ss], isem[1 - ss])
                pltpu.async_copy(dst_hbm.at[pl.ds(nrow, SUPC)],
                                 dbuf[1 - ss], isem[1 - ss])

    pltpu.make_async_copy(mb0, agg_sh.at[dbuf1.at[0]], ssem0).wait()
    pltpu.make_async_copy(mb1, agg_sh.at[dbuf1.at[0]], ssem1).wait()

    plsc.subcore_barrier()

    @pl.when(sid < NS - 1)
    def _():
      pltpu.sync_copy(agg_sh.at[pl.ds(row0, ROWS_A)],
                      agg_hbm.at[pl.ds(cid * N + row0, ROWS_A)])

    @pl.when(sid == NS - 1)
    def _():
      pltpu.sync_copy(agg_sh.at[pl.ds(row0, ROWS_LAST)],
                      agg_hbm.at[pl.ds(cid * N + row0, ROWS_LAST)])

  return k(h, exP, rden, srcP, dstP)


def _inv_perm():
  inv = np.zeros(HEADS * HID, np.int32)
  for r in range(HEADS * HID // 32):
    for i in range(16):
      for par in (0, 1):
        inv[r * 32 + 2 * i + par] = 32 * r + 16 * par + i
  return inv


_INV = _inv_perm()


def _block_diag(att):
  flat = att.reshape(-1)[_INV]
  r = jnp.arange(HEADS * HID)
  B = jnp.zeros((HEADS * HID, 16), jnp.float32)
  return B.at[r, _INV // HID].set(flat)


def _gat_layer(xin, W, att_src, att_dst, src40, dst40, srcP, dstP,
               skip_W=None, skip_b=None):
  Bs = _block_diag(att_src)
  Bd = _block_diag(att_dst)
  WP = W[:, _INV]
  if skip_W is not None:
    h, aas, aad, ms, md, skip = _dense_block(xin, WP, Bs, Bd, skip_W, skip_b)
  else:
    h, aas, aad, ms, md = _dense_block(xin, WP, Bs, Bd)
    skip = None
  m_raw = (ms + md).reshape(16)
  exP, den = _sc_pass1(aas, aad, m_raw, src40, dst40)
  rden = _den_recip(den)
  agg = _sc_pass2(h, exP, rden, srcP, dstP)
  return agg[:N], agg[N:], skip


def kernel(x, edge_index, batch_idx, W1, att_src1, att_dst1, bias1, bn1_g,
           bn1_b, skip_W, skip_b, W2, att_src2, att_dst2, bias2, bn2_g,
           bn2_b):
  src = edge_index[0].astype(jnp.int32)
  dst = edge_index[1].astype(jnp.int32)
  src40 = src.reshape(IR1, CH1)
  dst40 = dst.reshape(IR1, CH1)
  srcP = jnp.pad(src.reshape(NW, E_PER_W),
                 ((0, 0), (0, PAD_W))).reshape(EP // CHK, CHK)
  dstP = jnp.pad(dst.reshape(NW, E_PER_W),
                 ((0, 0), (0, PAD_W))).reshape(EP // CHK, CHK)

  p0, p1, skip = _gat_layer(x, W1, att_src1, att_dst1, src40, dst40,
                            srcP, dstP, skip_W, skip_b)
  hmid = _post_bn_gelu(p0, p1, skip, bias1, bn1_g, bn1_b)

  q0, q1, _ = _gat_layer(hmid, W2, att_src2, att_dst2, src40, dst40,
                         srcP, dstP)
  batch_row = batch_idx.astype(jnp.int32).reshape(GRID_N, 1, BN_BLK)
  return _post_bn_gelu_pool(q0, q1, batch_row, bias2, bn2_g, bn2_b)
